# Initial kernel scaffold; baseline (speedup 1.0000x reference)
import jax
import jax.numpy as jnp
from jax.experimental import pallas as pl


def kernel(x, edge_index, edge_attr, lin1_W, lin1_b, root1, bias1, lin2_W, lin2_b, root2, bias2, lin3_W, lin3_b, root3, bias3):
    raise NotImplementedError("write your pallas kernel here")


# speedup vs baseline: 1.5761x; 1.5761x over previous
import functools

import jax
import jax.numpy as jnp
from jax import lax
from jax.experimental import pallas as pl
from jax.experimental.pallas import tpu as pltpu
from jax.experimental.pallas import tpu_sc as plsc

N = 2048
E = 65536
NV = 16

NC = 2
NS = 16
NW = NC * NS
EPW = E // NW
CH = 128
NCH = EPW // CH
RPT = N // NS

@functools.lru_cache(maxsize=None)
def _mesh():
  return plsc.VectorSubcoreMesh(
      core_axis_name="c", subcore_axis_name="s", num_cores=NC, num_subcores=NS)


def _gather_body(table_hbm, src_hbm, out_hbm, idx_v, rows_v, sem):
  wid = lax.axis_index("s") * NC + lax.axis_index("c")
  pltpu.sync_copy(src_hbm.at[wid], idx_v)
  cps = []
  for ch in range(NCH):
    cps.append(pltpu.async_copy(
        table_hbm.at[idx_v.at[ch]], rows_v.at[pl.ds(ch * CH, CH)], sem))
  for cp in cps:
    cp.wait()
  pltpu.sync_copy(rows_v, out_hbm.at[pl.ds(wid * EPW, EPW)])


_SC_PARAMS = pltpu.CompilerParams(use_tc_tiling_on_sc=False)


def _make_gather(d):
  return pl.kernel(
      _gather_body,
      out_type=jax.ShapeDtypeStruct((E, d), jnp.float32),
      mesh=_mesh(),
      compiler_params=_SC_PARAMS,
      scratch_types=[
          pltpu.VMEM((NCH, CH), jnp.int32),
          pltpu.VMEM((EPW, d), jnp.float32),
          pltpu.SemaphoreType.DMA,
      ])


def _scatter_body(msg_hbm, dst_hbm, out_hbm, idx_v, msg_v, zbuf, accum, sem):
  cid = lax.axis_index("c")
  sid = lax.axis_index("s")
  wid = sid * NC + cid
  d = zbuf.shape[1]
  zv = jnp.zeros((16,), jnp.float32)
  for r in range(RPT):
    for c in range(d // 16):
      zbuf[r, pl.ds(c * 16, 16)] = zv
  pltpu.sync_copy(zbuf, accum.at[pl.ds(sid * RPT, RPT)])
  plsc.subcore_barrier()
  pltpu.sync_copy(dst_hbm.at[wid], idx_v)
  pltpu.sync_copy(msg_hbm.at[pl.ds(wid * EPW, EPW)], msg_v)
  for ch in range(NCH):
    pltpu.sync_copy(msg_v.at[pl.ds(ch * CH, CH)], accum.at[idx_v.at[ch]],
                    add=True)
  plsc.subcore_barrier()
  pltpu.sync_copy(accum.at[pl.ds(sid * RPT, RPT)],
                  out_hbm.at[cid, pl.ds(sid * RPT, RPT)])


def _make_scatter(d):
  return pl.kernel(
      _scatter_body,
      out_type=jax.ShapeDtypeStruct((NC, N, d), jnp.float32),
      mesh=_mesh(),
      compiler_params=_SC_PARAMS,
      scratch_types=[
          pltpu.VMEM((NCH, CH), jnp.int32),
          pltpu.VMEM((EPW, d), jnp.float32),
          pltpu.VMEM((RPT, d), jnp.float32),
          pltpu.VMEM_SHARED((N, d), jnp.float32),
          pltpu.SemaphoreType.DMA,
      ])


BE = 2048


def _msg1_body(a_ref, xj_ref, w_ref, b_ref, o_ref):
  w = jax.nn.relu(
      jnp.dot(a_ref[...], w_ref[...], preferred_element_type=jnp.float32)
      + b_ref[...])
  m = xj_ref[:, 0:1] * w
  ones = jnp.ones((BE, 1), jnp.float32)
  zeros = jnp.zeros((BE, 15), jnp.float32)
  o_ref[...] = jnp.concatenate([m, ones, zeros], axis=1)


def _msg2_body(a_ref, xj_ref, w_ref, b_ref, o_ref):
  w = jax.nn.relu(
      jnp.dot(a_ref[...], w_ref[...], preferred_element_type=jnp.float32)
      + b_ref[...])
  xj = xj_ref[...]
  acc = xj[:, 0:1] * w[:, 0:32]
  for i in range(1, 16):
    acc = acc + xj[:, i:i + 1] * w[:, i * 32:(i + 1) * 32]
  o_ref[...] = acc


def _msg3_body(a_ref, xj_ref, w_ref, b_ref, o_ref):
  w = jax.nn.relu(
      jnp.dot(a_ref[...], w_ref[...], preferred_element_type=jnp.float32)
      + b_ref[...])
  xj = xj_ref[...]
  acc = xj[:, 0:1] * w[:, 0:8]
  for i in range(1, 32):
    acc = acc + xj[:, i:i + 1] * w[:, i * 8:(i + 1) * 8]
  o_ref[...] = jnp.concatenate([acc, jnp.zeros((BE, 8), jnp.float32)], axis=1)


def _msg_call(body, c_in, kdim, dout):
  return pl.pallas_call(
      body,
      grid=(E // BE,),
      in_specs=[
          pl.BlockSpec((BE, NV), lambda i: (i, 0)),
          pl.BlockSpec((BE, c_in), lambda i: (i, 0)),
          pl.BlockSpec((NV, kdim), lambda i: (0, 0)),
          pl.BlockSpec((1, kdim), lambda i: (0, 0)),
      ],
      out_specs=pl.BlockSpec((BE, dout), lambda i: (i, 0)),
      out_shape=jax.ShapeDtypeStruct((E, dout), jnp.float32),
  )


def _combine1_body(p_ref, x_ref, r_ref, b_ref, h_ref, cnt_ref):
  s = p_ref[0, :, 0:16] + p_ref[1, :, 0:16]
  cnt = p_ref[0, :, 16:17] + p_ref[1, :, 16:17]
  cntc = jnp.maximum(cnt, 1.0)
  root = jnp.dot(x_ref[...], r_ref[...], preferred_element_type=jnp.float32)
  h_ref[...] = jax.nn.relu(s / cntc + root + b_ref[...])
  cnt_ref[...] = cntc


def _combine_body(p_ref, cnt_ref, h_ref, r_ref, b_ref, o_ref, dout):
  s = p_ref[0, :, 0:dout] + p_ref[1, :, 0:dout]
  root = jnp.dot(h_ref[...], r_ref[...], preferred_element_type=jnp.float32)
  o_ref[...] = jax.nn.relu(s / cnt_ref[...] + root + b_ref[...])


def _cbt_body(hblk_ref, ht_ref, o_ref):
  ha = hblk_ref[...]
  ht = ht_ref[...]
  acc = jnp.abs(ht[0:1, :] - ha[:, 0:1])
  for k in range(1, 8):
    acc = acc + jnp.abs(ht[k:k + 1, :] - ha[:, k:k + 1])
  o_ref[...] = acc


TA = 256


def kernel(x, edge_index, edge_attr, lin1_W, lin1_b, root1, bias1,
           lin2_W, lin2_b, root2, bias2, lin3_W, lin3_b, root3, bias3):
  f32 = jnp.float32
  src = edge_index[0].reshape(NW, NCH, CH)
  dst = edge_index[1].reshape(NW, NCH, CH)
  x16 = jnp.pad(x, ((0, 0), (0, 15)))

  gather16 = _make_gather(16)
  gather32 = _make_gather(32)
  scatter32 = _make_scatter(32)
  scatter16 = _make_scatter(16)

  xj1 = gather16(x16, src)
  msg1 = _msg_call(_msg1_body, 16, 16, 32)(
      edge_attr, xj1, lin1_W, lin1_b.reshape(1, 16))
  p1 = scatter32(msg1, dst)
  h1, cntc = pl.pallas_call(
      _combine1_body,
      in_specs=[pl.BlockSpec(p1.shape, lambda: (0, 0, 0)),
                pl.BlockSpec((N, 1), lambda: (0, 0)),
                pl.BlockSpec((1, 16), lambda: (0, 0)),
                pl.BlockSpec((1, 16), lambda: (0, 0))],
      out_specs=[pl.BlockSpec((N, 16), lambda: (0, 0)),
                 pl.BlockSpec((N, 1), lambda: (0, 0))],
      out_shape=[jax.ShapeDtypeStruct((N, 16), f32),
                 jax.ShapeDtypeStruct((N, 1), f32)],
  )(p1, x, root1, bias1.reshape(1, 16))

  xj2 = gather16(h1, src)
  msg2 = _msg_call(_msg2_body, 16, 16 * 32, 32)(
      edge_attr, xj2, lin2_W, lin2_b.reshape(1, 16 * 32))
  p2 = scatter32(msg2, dst)
  h2 = pl.pallas_call(
      functools.partial(_combine_body, dout=32),
      in_specs=[pl.BlockSpec(p2.shape, lambda: (0, 0, 0)),
                pl.BlockSpec((N, 1), lambda: (0, 0)),
                pl.BlockSpec((N, 16), lambda: (0, 0)),
                pl.BlockSpec((16, 32), lambda: (0, 0)),
                pl.BlockSpec((1, 32), lambda: (0, 0))],
      out_specs=pl.BlockSpec((N, 32), lambda: (0, 0)),
      out_shape=jax.ShapeDtypeStruct((N, 32), f32),
  )(p2, cntc, h1, root2, bias2.reshape(1, 32))

  xj3 = gather32(h2, src)
  msg3 = _msg_call(_msg3_body, 32, 16 * 16, 16)(
      edge_attr, xj3, lin3_W, lin3_b.reshape(1, 16 * 16))
  p3 = scatter16(msg3, dst)
  h3, h3t = pl.pallas_call(
      _combine3_body,
      in_specs=[pl.BlockSpec(p3.shape, lambda: (0, 0, 0)),
                pl.BlockSpec((N, 1), lambda: (0, 0)),
                pl.BlockSpec((N, 32), lambda: (0, 0)),
                pl.BlockSpec((32, 8), lambda: (0, 0)),
                pl.BlockSpec((1, 8), lambda: (0, 0))],
      out_specs=[pl.BlockSpec((N, 8), lambda: (0, 0)),
                 pl.BlockSpec((8, N), lambda: (0, 0))],
      out_shape=[jax.ShapeDtypeStruct((N, 8), f32),
                 jax.ShapeDtypeStruct((8, N), f32)],
  )(p3, cntc, h2, root3, bias3.reshape(1, 8))

  cbt = pl.pallas_call(
      _cbt_body,
      grid=(N // TA,),
      in_specs=[pl.BlockSpec((TA, 8), lambda i: (i, 0)),
                pl.BlockSpec((8, N), lambda i: (0, 0))],
      out_specs=pl.BlockSpec((TA, N), lambda i: (i, 0)),
      out_shape=jax.ShapeDtypeStruct((N, N), f32),
  )(h3, h3t)
  return cbt


def _combine3_body(p_ref, cnt_ref, h_ref, r_ref, b_ref, o_ref, ot_ref):
  s = p_ref[0, :, 0:8] + p_ref[1, :, 0:8]
  root = jnp.dot(h_ref[...], r_ref[...], preferred_element_type=jnp.float32)
  h3 = jax.nn.relu(s / cnt_ref[...] + root + b_ref[...])
  o_ref[...] = h3
  ot_ref[...] = h3.T


# speedup vs baseline: 4.3569x; 2.7644x over previous
import functools

import jax
import jax.numpy as jnp
from jax import lax
from jax.experimental import pallas as pl
from jax.experimental.pallas import tpu as pltpu
from jax.experimental.pallas import tpu_sc as plsc

N = 2048
E = 65536
NV = 16

NC = 2
NS = 16
NW = NC * NS
EPW = E // NW
CH = 128
NCH = EPW // CH
RPT = N // NS

@functools.lru_cache(maxsize=None)
def _mesh():
  return plsc.VectorSubcoreMesh(
      core_axis_name="c", subcore_axis_name="s", num_cores=NC, num_subcores=NS)


def _gather_body(table_hbm, src_hbm, out_hbm, idx_v, rows_v, sem):
  wid = lax.axis_index("s") * NC + lax.axis_index("c")
  pltpu.sync_copy(src_hbm.at[wid], idx_v)
  cps = []
  for ch in range(NCH):
    cps.append(pltpu.async_copy(
        table_hbm.at[idx_v.at[ch]], rows_v.at[pl.ds(ch * CH, CH)], sem))
  for cp in cps:
    cp.wait()
  pltpu.sync_copy(rows_v, out_hbm.at[pl.ds(wid * EPW, EPW)])


_SC_PARAMS = pltpu.CompilerParams(use_tc_tiling_on_sc=False)


def _make_gather(d):
  return pl.kernel(
      _gather_body,
      out_type=jax.ShapeDtypeStruct((E, d), jnp.float32),
      mesh=_mesh(),
      compiler_params=_SC_PARAMS,
      scratch_types=[
          pltpu.VMEM((NCH, CH), jnp.int32),
          pltpu.VMEM((EPW, d), jnp.float32),
          pltpu.SemaphoreType.DMA,
      ])


def _scatter_body(msg_hbm, dst_hbm, out_hbm, idx_v, msg_v, zbuf, accum, sem):
  cid = lax.axis_index("c")
  sid = lax.axis_index("s")
  wid = sid * NC + cid
  d = zbuf.shape[1]
  zv = jnp.zeros((16,), jnp.float32)
  for r in range(RPT):
    for c in range(d // 16):
      zbuf[r, pl.ds(c * 16, 16)] = zv
  pltpu.sync_copy(zbuf, accum.at[pl.ds(sid * RPT, RPT)])
  plsc.subcore_barrier()
  pltpu.sync_copy(dst_hbm.at[wid], idx_v)
  pltpu.sync_copy(msg_hbm.at[pl.ds(wid * EPW, EPW)], msg_v)
  for ch in range(NCH):
    pltpu.sync_copy(msg_v.at[pl.ds(ch * CH, CH)], accum.at[idx_v.at[ch]],
                    add=True)
  plsc.subcore_barrier()
  pltpu.sync_copy(accum.at[pl.ds(sid * RPT, RPT)],
                  out_hbm.at[cid, pl.ds(sid * RPT, RPT)])


def _make_scatter(d):
  return pl.kernel(
      _scatter_body,
      out_type=jax.ShapeDtypeStruct((NC, N, d), jnp.float32),
      mesh=_mesh(),
      compiler_params=_SC_PARAMS,
      scratch_types=[
          pltpu.VMEM((NCH, CH), jnp.int32),
          pltpu.VMEM((EPW, d), jnp.float32),
          pltpu.VMEM((RPT, d), jnp.float32),
          pltpu.VMEM_SHARED((N, d), jnp.float32),
          pltpu.SemaphoreType.DMA,
      ])


BE = 2048


def _msg1_body(a_ref, xj_ref, w_ref, o_ref):
  aug = jnp.concatenate([a_ref[...], jnp.ones((BE, 1), jnp.float32)], axis=1)
  w = jax.nn.relu(
      jnp.dot(aug, w_ref[...], preferred_element_type=jnp.float32))
  m = xj_ref[:, 0:1] * w
  ones = jnp.ones((BE, 1), jnp.float32)
  zeros = jnp.zeros((BE, 15), jnp.float32)
  o_ref[...] = jnp.concatenate([m, ones, zeros], axis=1)


def _expand_contract(c_in, c_out, kdim):
  ri = lax.broadcasted_iota(jnp.int32, (c_in, kdim), 0)
  rc = lax.broadcasted_iota(jnp.int32, (c_in, kdim), 1)
  r = (rc // c_out == ri).astype(jnp.float32)
  si = lax.broadcasted_iota(jnp.int32, (kdim, c_out), 0)
  so = lax.broadcasted_iota(jnp.int32, (kdim, c_out), 1)
  s = (si % c_out == so).astype(jnp.float32)
  return r, s


def _msg2_body(a_ref, xj_ref, w_ref, o_ref):
  aug = jnp.concatenate([a_ref[...], jnp.ones((BE, 1), jnp.float32)], axis=1)
  w = jax.nn.relu(
      jnp.dot(aug, w_ref[...], preferred_element_type=jnp.float32))
  r, s = _expand_contract(16, 32, 512)
  xr = jnp.dot(xj_ref[...], r, preferred_element_type=jnp.float32)
  o_ref[...] = jnp.dot(xr * w, s, preferred_element_type=jnp.float32)


def _msg3_body(a_ref, xj_ref, w_ref, o_ref):
  aug = jnp.concatenate([a_ref[...], jnp.ones((BE, 1), jnp.float32)], axis=1)
  w = jax.nn.relu(
      jnp.dot(aug, w_ref[...], preferred_element_type=jnp.float32))
  r, _ = _expand_contract(32, 8, 256)
  si = lax.broadcasted_iota(jnp.int32, (256, 16), 0)
  so = lax.broadcasted_iota(jnp.int32, (256, 16), 1)
  s = (si % 8 == so).astype(jnp.float32)
  xr = jnp.dot(xj_ref[...], r, preferred_element_type=jnp.float32)
  o_ref[...] = jnp.dot(xr * w, s, preferred_element_type=jnp.float32)


def _msg_call(body, c_in, kdim, dout, nw_rows=NV):
  return pl.pallas_call(
      body,
      grid=(E // BE,),
      in_specs=[
          pl.BlockSpec((BE, NV), lambda i: (i, 0)),
          pl.BlockSpec((BE, c_in), lambda i: (i, 0)),
          pl.BlockSpec((nw_rows, kdim), lambda i: (0, 0)),
      ],
      out_specs=pl.BlockSpec((BE, dout), lambda i: (i, 0)),
      out_shape=jax.ShapeDtypeStruct((E, dout), jnp.float32),
  )


def _combine1_body(p_ref, x_ref, r_ref, b_ref, h_ref, cnt_ref):
  s = p_ref[0, :, 0:16] + p_ref[1, :, 0:16]
  cnt = p_ref[0, :, 16:17] + p_ref[1, :, 16:17]
  cntc = jnp.maximum(cnt, 1.0)
  root = jnp.dot(x_ref[...], r_ref[...], preferred_element_type=jnp.float32)
  h_ref[...] = jax.nn.relu(s / cntc + root + b_ref[...])
  cnt_ref[...] = cntc


def _combine_body(p_ref, cnt_ref, h_ref, r_ref, b_ref, o_ref, dout):
  s = p_ref[0, :, 0:dout] + p_ref[1, :, 0:dout]
  root = jnp.dot(h_ref[...], r_ref[...], preferred_element_type=jnp.float32)
  o_ref[...] = jax.nn.relu(s / cnt_ref[...] + root + b_ref[...])


def _cbt_body(hblk_ref, ht_ref, o_ref):
  ha = hblk_ref[...]
  ht = ht_ref[...]
  acc = jnp.abs(ht[0:1, :] - ha[:, 0:1])
  for k in range(1, 8):
    acc = acc + jnp.abs(ht[k:k + 1, :] - ha[:, k:k + 1])
  o_ref[...] = acc


TA = 256


def kernel(x, edge_index, edge_attr, lin1_W, lin1_b, root1, bias1,
           lin2_W, lin2_b, root2, bias2, lin3_W, lin3_b, root3, bias3):
  f32 = jnp.float32
  src = edge_index[0].reshape(NW, NCH, CH)
  dst = edge_index[1].reshape(NW, NCH, CH)
  x16 = jnp.pad(x, ((0, 0), (0, 15)))

  gather16 = _make_gather(16)
  gather32 = _make_gather(32)
  scatter32 = _make_scatter(32)
  scatter16 = _make_scatter(16)

  xj1 = gather16(x16, src)
  msg1 = _msg_call(_msg1_body, 16, 16, 32, nw_rows=17)(
      edge_attr, xj1, jnp.concatenate([lin1_W, lin1_b[None, :]], axis=0))
  p1 = scatter32(msg1, dst)
  h1, cntc = pl.pallas_call(
      _combine1_body,
      in_specs=[pl.BlockSpec(p1.shape, lambda: (0, 0, 0)),
                pl.BlockSpec((N, 1), lambda: (0, 0)),
                pl.BlockSpec((1, 16), lambda: (0, 0)),
                pl.BlockSpec((1, 16), lambda: (0, 0))],
      out_specs=[pl.BlockSpec((N, 16), lambda: (0, 0)),
                 pl.BlockSpec((N, 1), lambda: (0, 0))],
      out_shape=[jax.ShapeDtypeStruct((N, 16), f32),
                 jax.ShapeDtypeStruct((N, 1), f32)],
  )(p1, x, root1, bias1.reshape(1, 16))

  xj2 = gather16(h1, src)
  msg2 = _msg_call(_msg2_body, 16, 16 * 32, 32, nw_rows=17)(
      edge_attr, xj2, jnp.concatenate([lin2_W, lin2_b[None, :]], axis=0))
  p2 = scatter32(msg2, dst)
  h2 = pl.pallas_call(
      functools.partial(_combine_body, dout=32),
      in_specs=[pl.BlockSpec(p2.shape, lambda: (0, 0, 0)),
                pl.BlockSpec((N, 1), lambda: (0, 0)),
                pl.BlockSpec((N, 16), lambda: (0, 0)),
                pl.BlockSpec((16, 32), lambda: (0, 0)),
                pl.BlockSpec((1, 32), lambda: (0, 0))],
      out_specs=pl.BlockSpec((N, 32), lambda: (0, 0)),
      out_shape=jax.ShapeDtypeStruct((N, 32), f32),
  )(p2, cntc, h1, root2, bias2.reshape(1, 32))

  xj3 = gather32(h2, src)
  msg3 = _msg_call(_msg3_body, 32, 16 * 16, 16, nw_rows=17)(
      edge_attr, xj3, jnp.concatenate([lin3_W, lin3_b[None, :]], axis=0))
  p3 = scatter16(msg3, dst)
  h3, h3t = pl.pallas_call(
      _combine3_body,
      in_specs=[pl.BlockSpec(p3.shape, lambda: (0, 0, 0)),
                pl.BlockSpec((N, 1), lambda: (0, 0)),
                pl.BlockSpec((N, 32), lambda: (0, 0)),
                pl.BlockSpec((32, 8), lambda: (0, 0)),
                pl.BlockSpec((1, 8), lambda: (0, 0))],
      out_specs=[pl.BlockSpec((N, 8), lambda: (0, 0)),
                 pl.BlockSpec((8, N), lambda: (0, 0))],
      out_shape=[jax.ShapeDtypeStruct((N, 8), f32),
                 jax.ShapeDtypeStruct((8, N), f32)],
  )(p3, cntc, h2, root3, bias3.reshape(1, 8))

  cbt = pl.pallas_call(
      _cbt_body,
      grid=(N // TA,),
      in_specs=[pl.BlockSpec((TA, 8), lambda i: (i, 0)),
                pl.BlockSpec((8, N), lambda i: (0, 0))],
      out_specs=pl.BlockSpec((TA, N), lambda i: (i, 0)),
      out_shape=jax.ShapeDtypeStruct((N, N), f32),
  )(h3, h3t)
  return cbt


def _combine3_body(p_ref, cnt_ref, h_ref, r_ref, b_ref, o_ref, ot_ref):
  s = p_ref[0, :, 0:8] + p_ref[1, :, 0:8]
  root = jnp.dot(h_ref[...], r_ref[...], preferred_element_type=jnp.float32)
  h3 = jax.nn.relu(s / cnt_ref[...] + root + b_ref[...])
  o_ref[...] = h3
  ot_ref[...] = h3.T


# speedup vs baseline: 5.1791x; 1.1887x over previous
import functools

import jax
import jax.numpy as jnp
from jax import lax
from jax.experimental import pallas as pl
from jax.experimental.pallas import tpu as pltpu
from jax.experimental.pallas import tpu_sc as plsc

N = 2048
E = 65536
NV = 16

NC = 2
NS = 16
NW = NC * NS
EPW = E // NW
CH = 128
NCH = EPW // CH
RPT = N // NS

BE = 2048
G = 8


@functools.lru_cache(maxsize=None)
def _mesh():
  return plsc.VectorSubcoreMesh(
      core_axis_name="c", subcore_axis_name="s", num_cores=NC, num_subcores=NS)


_SC_PARAMS = pltpu.CompilerParams(use_tc_tiling_on_sc=False)


def _gather_body(table_hbm, src_hbm, out_hbm, idx_v, rows_v, sem):
  wid = lax.axis_index("s") * NC + lax.axis_index("c")
  pltpu.sync_copy(src_hbm.at[wid], idx_v)
  cps = []
  for ch in range(NCH):
    cps.append(pltpu.async_copy(
        table_hbm.at[idx_v.at[ch]], rows_v.at[pl.ds(ch * CH, CH)], sem))
  for cp in cps:
    cp.wait()
  pltpu.sync_copy(rows_v, out_hbm.at[pl.ds(wid * EPW, EPW)])


def _make_gather(d):
  return pl.kernel(
      _gather_body,
      out_type=jax.ShapeDtypeStruct((E, d), jnp.float32),
      mesh=_mesh(),
      compiler_params=_SC_PARAMS,
      scratch_types=[
          pltpu.VMEM((NCH, CH), jnp.int32),
          pltpu.VMEM((EPW, d), jnp.float32),
          pltpu.SemaphoreType.DMA,
      ])


def _scatter_body(msg_hbm, dst_hbm, out_hbm, idx_v, msg_v, zbuf, accum, sem):
  cid = lax.axis_index("c")
  sid = lax.axis_index("s")
  wid = sid * NC + cid
  d = zbuf.shape[1]
  zv = jnp.zeros((16,), jnp.float32)
  for r in range(RPT):
    for c in range(d // 16):
      zbuf[r, pl.ds(c * 16, 16)] = zv
  pltpu.sync_copy(zbuf, accum.at[pl.ds(sid * RPT, RPT)])
  plsc.subcore_barrier()
  pltpu.sync_copy(dst_hbm.at[wid], idx_v)
  pltpu.sync_copy(msg_hbm.at[pl.ds(wid * EPW, EPW)], msg_v)
  for ch in range(NCH):
    pltpu.sync_copy(msg_v.at[pl.ds(ch * CH, CH)], accum.at[idx_v.at[ch]],
                    add=True)
  plsc.subcore_barrier()
  pltpu.sync_copy(accum.at[pl.ds(sid * RPT, RPT)],
                  out_hbm.at[cid, pl.ds(sid * RPT, RPT)])


def _make_scatter(d):
  return pl.kernel(
      _scatter_body,
      out_type=jax.ShapeDtypeStruct((NC, N, d), jnp.float32),
      mesh=_mesh(),
      compiler_params=_SC_PARAMS,
      scratch_types=[
          pltpu.VMEM((NCH, CH), jnp.int32),
          pltpu.VMEM((EPW, d), jnp.float32),
          pltpu.VMEM((RPT, d), jnp.float32),
          pltpu.VMEM_SHARED((N, d), jnp.float32),
          pltpu.SemaphoreType.DMA,
      ])


R256 = BE // G


def _expand_mat(c_in, c_out):
  kdim = c_in * c_out
  ri = lax.broadcasted_iota(jnp.int32, (c_in, kdim), 0)
  rc = lax.broadcasted_iota(jnp.int32, (c_in, kdim), 1)
  return (rc // c_out == ri).astype(jnp.float32)


def _tree_contract(acc, c_out):
  width = acc.shape[1]
  while width > c_out:
    width //= 2
    acc = acc[:, :width] + acc[:, width:]
  return acc


def _unpack_slots(pk, c):
  if c * 4 == 128:
    slots = [pk[(g // 4) * R256:(g // 4 + 1) * R256,
                (g % 4) * c:(g % 4 + 1) * c] for g in range(G)]
  else:
    slots = [pk[:, g * c:(g + 1) * c] for g in range(G)]
  return jnp.concatenate(slots, axis=0)


def _slot_msg(a_pk, xj_pk, waug, c_in, c_out):
  r = _expand_mat(c_in, c_out)
  a_all = _unpack_slots(a_pk, NV)
  xj_all = _unpack_slots(xj_pk, c_in)
  aug = jnp.concatenate([a_all, jnp.ones((BE, 1), jnp.float32)], axis=1)
  w = jax.nn.relu(
      jnp.dot(aug, waug, preferred_element_type=jnp.float32))
  xr = jnp.dot(xj_all, r, preferred_element_type=jnp.float32)
  return _tree_contract(xr * w, c_out)


def _assemble32(m_all):
  ms = [m_all[g * R256:(g + 1) * R256] for g in range(G)]
  top = jnp.concatenate(ms[0:4], axis=1)
  bot = jnp.concatenate(ms[4:8], axis=1)
  return jnp.concatenate([top, bot], axis=0)


def _msg1_body(a_ref, xj_ref, w_ref, o_ref):
  a_all = _unpack_slots(a_ref[...], NV)
  xj_all = _unpack_slots(xj_ref[...], NV)
  aug = jnp.concatenate([a_all, jnp.ones((BE, 1), jnp.float32)], axis=1)
  w = jax.nn.relu(
      jnp.dot(aug, w_ref[...], preferred_element_type=jnp.float32))
  m = xj_all[:, 0:1] * w
  msg = jnp.concatenate(
      [m, jnp.ones((BE, 1), jnp.float32), jnp.zeros((BE, 15), jnp.float32)],
      axis=1)
  o_ref[...] = _assemble32(msg)


def _msg2_body(a_ref, xj_ref, w_ref, o_ref):
  o_ref[...] = _assemble32(
      _slot_msg(a_ref[...], xj_ref[...], w_ref[...], 16, 32))


def _msg3_body(a_ref, xj_ref, w_ref, o_ref):
  m_all = _slot_msg(a_ref[...], xj_ref[...], w_ref[...], 32, 8)
  padded = jnp.concatenate(
      [m_all, jnp.zeros((BE, 8), jnp.float32)], axis=1)
  o_ref[...] = jnp.concatenate(
      [padded[g * R256:(g + 1) * R256] for g in range(G)], axis=1)


def _msg_call(body, xin_rows, out_rows, kdim):
  bxin = xin_rows // (E // BE)
  bout = out_rows // (E // BE)
  return pl.pallas_call(
      body,
      grid=(E // BE,),
      in_specs=[
          pl.BlockSpec((R256, 128), lambda i: (i, 0)),
          pl.BlockSpec((bxin, 128), lambda i: (i, 0)),
          pl.BlockSpec((17, kdim), lambda i: (0, 0)),
      ],
      out_specs=pl.BlockSpec((bout, 128), lambda i: (i, 0)),
      out_shape=jax.ShapeDtypeStruct((out_rows, 128), jnp.float32),
  )


def _combine1_body(p_ref, x_ref, r_ref, b_ref, h_ref, cnt_ref):
  s = p_ref[0, :, 0:16] + p_ref[1, :, 0:16]
  cnt = p_ref[0, :, 16:17] + p_ref[1, :, 16:17]
  cntc = jnp.maximum(cnt, 1.0)
  root = jnp.dot(x_ref[...], r_ref[...], preferred_element_type=jnp.float32)
  h_ref[...] = jax.nn.relu(s / cntc + root + b_ref[...])
  cnt_ref[...] = cntc


def _combine_body(p_ref, cnt_ref, h_ref, r_ref, b_ref, o_ref, dout):
  s = p_ref[0, :, 0:dout] + p_ref[1, :, 0:dout]
  root = jnp.dot(h_ref[...], r_ref[...], preferred_element_type=jnp.float32)
  o_ref[...] = jax.nn.relu(s / cnt_ref[...] + root + b_ref[...])


def _combine3_body(p_ref, cnt_ref, h_ref, r_ref, b_ref, o_ref, ot_ref):
  s = p_ref[0, :, 0:8] + p_ref[1, :, 0:8]
  root = jnp.dot(h_ref[...], r_ref[...], preferred_element_type=jnp.float32)
  h3 = jax.nn.relu(s / cnt_ref[...] + root + b_ref[...])
  o_ref[...] = h3
  ot_ref[...] = h3.T


def _cbt_body(hblk_ref, ht_ref, o_ref):
  ha = hblk_ref[...]
  ht = ht_ref[...]
  acc = jnp.abs(ht[0:1, :] - ha[:, 0:1])
  for k in range(1, 8):
    acc = acc + jnp.abs(ht[k:k + 1, :] - ha[:, k:k + 1])
  o_ref[...] = acc


TA = 256


def _kperm(arr):
  return arr.reshape(E // BE, R256, 2, 4).transpose(0, 2, 1, 3)


def kernel(x, edge_index, edge_attr, lin1_W, lin1_b, root1, bias1,
           lin2_W, lin2_b, root2, bias2, lin3_W, lin3_b, root3, bias3):
  f32 = jnp.float32
  src = edge_index[0].reshape(NW, NCH, CH)
  dst = edge_index[1].reshape(NW, NCH, CH)
  dst_k = _kperm(edge_index[1]).reshape(NW, NCH, CH)
  src_k = _kperm(edge_index[0]).reshape(NW, NCH, CH)
  x16 = jnp.pad(x, ((0, 0), (0, 15)))
  ap = edge_attr.reshape(E // G, 128)

  gather16 = _make_gather(16)
  gather32 = _make_gather(32)
  scatter32 = _make_scatter(32)
  scatter16 = _make_scatter(16)

  waug1 = jnp.concatenate([lin1_W, lin1_b[None, :]], axis=0)
  waug2 = jnp.concatenate([lin2_W, lin2_b[None, :]], axis=0)
  waug3 = jnp.concatenate([lin3_W, lin3_b[None, :]], axis=0)

  xj1 = gather16(x16, src)
  msg1 = _msg_call(_msg1_body, E // 8, E // 4, 16)(
      ap, xj1.reshape(E // 8, 128), waug1)
  p1 = scatter32(msg1.reshape(E, 32), dst_k)
  h1, cntc = pl.pallas_call(
      _combine1_body,
      in_specs=[pl.BlockSpec((NC, N, 32), lambda: (0, 0, 0)),
                pl.BlockSpec((N, 1), lambda: (0, 0)),
                pl.BlockSpec((1, 16), lambda: (0, 0)),
                pl.BlockSpec((1, 16), lambda: (0, 0))],
      out_specs=[pl.BlockSpec((N, 16), lambda: (0, 0)),
                 pl.BlockSpec((N, 1), lambda: (0, 0))],
      out_shape=[jax.ShapeDtypeStruct((N, 16), f32),
                 jax.ShapeDtypeStruct((N, 1), f32)],
  )(p1, x, root1, bias1.reshape(1, 16))

  xj2 = gather16(h1, src)
  msg2 = _msg_call(_msg2_body, E // 8, E // 4, 512)(
      ap, xj2.reshape(E // 8, 128), waug2)
  p2 = scatter32(msg2.reshape(E, 32), dst_k)
  h2 = pl.pallas_call(
      functools.partial(_combine_body, dout=32),
      in_specs=[pl.BlockSpec((NC, N, 32), lambda: (0, 0, 0)),
                pl.BlockSpec((N, 1), lambda: (0, 0)),
                pl.BlockSpec((N, 16), lambda: (0, 0)),
                pl.BlockSpec((16, 32), lambda: (0, 0)),
                pl.BlockSpec((1, 32), lambda: (0, 0))],
      out_specs=pl.BlockSpec((N, 32), lambda: (0, 0)),
      out_shape=jax.ShapeDtypeStruct((N, 32), f32),
  )(p2, cntc, h1, root2, bias2.reshape(1, 32))

  xj3 = gather32(h2, src_k)
  msg3 = _msg_call(_msg3_body, E // 4, E // 8, 256)(
      ap, xj3.reshape(E // 4, 128), waug3)
  p3 = scatter16(msg3.reshape(E, 16), dst)
  h3, h3t = pl.pallas_call(
      _combine3_body,
      in_specs=[pl.BlockSpec((NC, N, 16), lambda: (0, 0, 0)),
                pl.BlockSpec((N, 1), lambda: (0, 0)),
                pl.BlockSpec((N, 32), lambda: (0, 0)),
                pl.BlockSpec((32, 8), lambda: (0, 0)),
                pl.BlockSpec((1, 8), lambda: (0, 0))],
      out_specs=[pl.BlockSpec((N, 8), lambda: (0, 0)),
                 pl.BlockSpec((8, N), lambda: (0, 0))],
      out_shape=[jax.ShapeDtypeStruct((N, 8), f32),
                 jax.ShapeDtypeStruct((8, N), f32)],
  )(p3, cntc, h2, root3, bias3.reshape(1, 8))

  cbt = pl.pallas_call(
      _cbt_body,
      grid=(N // TA,),
      in_specs=[pl.BlockSpec((TA, 8), lambda i: (i, 0)),
                pl.BlockSpec((8, N), lambda i: (0, 0))],
      out_specs=pl.BlockSpec((TA, N), lambda i: (i, 0)),
      out_shape=jax.ShapeDtypeStruct((N, N), f32),
  )(h3, h3t)
  return cbt


# speedup vs baseline: 5.3639x; 1.0357x over previous
import functools

import jax
import jax.numpy as jnp
from jax import lax
from jax.experimental import pallas as pl
from jax.experimental.pallas import tpu as pltpu
from jax.experimental.pallas import tpu_sc as plsc

N = 2048
E = 65536
NV = 16

NC = 2
NS = 16
NW = NC * NS
EPW = E // NW
CH = 128
NCH = EPW // CH
RPT = N // NS

BE = 2048
G = 8


@functools.lru_cache(maxsize=None)
def _mesh():
  return plsc.VectorSubcoreMesh(
      core_axis_name="c", subcore_axis_name="s", num_cores=NC, num_subcores=NS)


_SC_PARAMS = pltpu.CompilerParams(use_tc_tiling_on_sc=False)


def _gather_work(table_hbm, out_hbm, idx_v, rows_v, sem, wid):
  cps = []
  for ch in range(NCH):
    cps.append(pltpu.async_copy(
        table_hbm.at[idx_v.at[ch]], rows_v.at[pl.ds(ch * CH, CH)], sem))
  for cp in cps:
    cp.wait()
  pltpu.sync_copy(rows_v, out_hbm.at[pl.ds(wid * EPW, EPW)])


def _gather16_body(table_hbm, src_hbm, out_hbm, idx_v, rows_v, sem):
  wid = lax.axis_index("s") * NC + lax.axis_index("c")
  pltpu.sync_copy(src_hbm.at[wid], idx_v)
  _gather_work(table_hbm, out_hbm, idx_v, rows_v, sem, wid)


def _make_gather16():
  return pl.kernel(
      _gather16_body,
      out_type=jax.ShapeDtypeStruct((E, 16), jnp.float32),
      mesh=_mesh(),
      compiler_params=_SC_PARAMS,
      scratch_types=[
          pltpu.VMEM((NCH, CH), jnp.int32),
          pltpu.VMEM((EPW, 16), jnp.float32),
          pltpu.SemaphoreType.DMA,
      ])


def _gather_pair_body(tlo_hbm, thi_hbm, src_hbm, olo_hbm, ohi_hbm,
                      idx_v, rlo_v, rhi_v, sem):
  wid = lax.axis_index("s") * NC + lax.axis_index("c")
  pltpu.sync_copy(src_hbm.at[wid], idx_v)
  _gather_work(tlo_hbm, olo_hbm, idx_v, rlo_v, sem, wid)
  _gather_work(thi_hbm, ohi_hbm, idx_v, rhi_v, sem, wid)


def _make_gather_pair():
  out16 = jax.ShapeDtypeStruct((E, 16), jnp.float32)
  return pl.kernel(
      _gather_pair_body,
      out_type=[out16, out16],
      mesh=_mesh(),
      compiler_params=_SC_PARAMS,
      scratch_types=[
          pltpu.VMEM((NCH, CH), jnp.int32),
          pltpu.VMEM((EPW, 16), jnp.float32),
          pltpu.VMEM((EPW, 16), jnp.float32),
          pltpu.SemaphoreType.DMA,
      ])


def _zero_accum(zbuf, accum, sid):
  zv = jnp.zeros((16,), jnp.float32)
  for r in range(RPT):
    zbuf[r, pl.ds(0, 16)] = zv
  pltpu.sync_copy(zbuf, accum.at[pl.ds(sid * RPT, RPT)])


def _scatter_stream(msg_hbm, idx_v, msg_v, accum, wid):
  pltpu.sync_copy(msg_hbm.at[pl.ds(wid * EPW, EPW)], msg_v)
  for ch in range(NCH):
    pltpu.sync_copy(msg_v.at[pl.ds(ch * CH, CH)], accum.at[idx_v.at[ch]],
                    add=True)


def _scatter16_body(msg_hbm, dst_hbm, out_hbm, idx_v, msg_v, zbuf, accum,
                    sem):
  cid = lax.axis_index("c")
  sid = lax.axis_index("s")
  wid = sid * NC + cid
  _zero_accum(zbuf, accum, sid)
  plsc.subcore_barrier()
  pltpu.sync_copy(dst_hbm.at[wid], idx_v)
  _scatter_stream(msg_hbm, idx_v, msg_v, accum, wid)
  plsc.subcore_barrier()
  pltpu.sync_copy(accum.at[pl.ds(sid * RPT, RPT)],
                  out_hbm.at[cid, pl.ds(sid * RPT, RPT)])


def _make_scatter16():
  return pl.kernel(
      _scatter16_body,
      out_type=jax.ShapeDtypeStruct((NC, N, 16), jnp.float32),
      mesh=_mesh(),
      compiler_params=_SC_PARAMS,
      scratch_types=[
          pltpu.VMEM((NCH, CH), jnp.int32),
          pltpu.VMEM((EPW, 16), jnp.float32),
          pltpu.VMEM((RPT, 16), jnp.float32),
          pltpu.VMEM_SHARED((N, 16), jnp.float32),
          pltpu.SemaphoreType.DMA,
      ])


def _scatter_pair_body(mlo_hbm, mhi_hbm, dst_hbm, out_hbm,
                       idx_v, vlo, vhi, zbuf, alo, ahi, sem):
  cid = lax.axis_index("c")
  sid = lax.axis_index("s")
  wid = sid * NC + cid
  _zero_accum(zbuf, alo, sid)
  _zero_accum(zbuf, ahi, sid)
  plsc.subcore_barrier()
  pltpu.sync_copy(dst_hbm.at[wid], idx_v)
  _scatter_stream(mlo_hbm, idx_v, vlo, alo, wid)
  _scatter_stream(mhi_hbm, idx_v, vhi, ahi, wid)
  plsc.subcore_barrier()
  pltpu.sync_copy(alo.at[pl.ds(sid * RPT, RPT)],
                  out_hbm.at[cid, 0, pl.ds(sid * RPT, RPT)])
  pltpu.sync_copy(ahi.at[pl.ds(sid * RPT, RPT)],
                  out_hbm.at[cid, 1, pl.ds(sid * RPT, RPT)])


def _make_scatter_pair():
  return pl.kernel(
      _scatter_pair_body,
      out_type=jax.ShapeDtypeStruct((NC, 2, N, 16), jnp.float32),
      mesh=_mesh(),
      compiler_params=_SC_PARAMS,
      scratch_types=[
          pltpu.VMEM((NCH, CH), jnp.int32),
          pltpu.VMEM((EPW, 16), jnp.float32),
          pltpu.VMEM((EPW, 16), jnp.float32),
          pltpu.VMEM((RPT, 16), jnp.float32),
          pltpu.VMEM_SHARED((N, 16), jnp.float32),
          pltpu.VMEM_SHARED((N, 16), jnp.float32),
          pltpu.SemaphoreType.DMA,
      ])


R256 = BE // G


def _expand_mat(c_in, c_out):
  kdim = c_in * c_out
  ri = lax.broadcasted_iota(jnp.int32, (c_in, kdim), 0)
  rc = lax.broadcasted_iota(jnp.int32, (c_in, kdim), 1)
  return (rc // c_out == ri).astype(jnp.float32)


def _tree_contract(acc, c_out):
  width = acc.shape[1]
  while width > c_out:
    width //= 2
    acc = acc[:, :width] + acc[:, width:]
  return acc


def _unpack_slots(pk):
  return jnp.concatenate([pk[:, g * 16:(g + 1) * 16] for g in range(G)],
                         axis=0)


def _assemble16(m_all):
  return jnp.concatenate([m_all[g * R256:(g + 1) * R256] for g in range(G)],
                         axis=1)


def _mlp_w(a_pk, waug):
  a_all = _unpack_slots(a_pk)
  aug = jnp.concatenate([a_all, jnp.ones((BE, 1), jnp.float32)], axis=1)
  return jax.nn.relu(
      jnp.dot(aug, waug, preferred_element_type=jnp.float32))


def _msg1_body(a_ref, xj_ref, w_ref, olo_ref, ohi_ref):
  w = _mlp_w(a_ref[...], w_ref[...])
  xj_all = _unpack_slots(xj_ref[...])
  m = xj_all[:, 0:1] * w
  olo_ref[...] = _assemble16(m)
  lanes = lax.broadcasted_iota(jnp.int32, (R256, 128), 1)
  ohi_ref[...] = (lanes % 16 == 0).astype(jnp.float32)


def _msg2_body(a_ref, xj_ref, w_ref, olo_ref, ohi_ref):
  w = _mlp_w(a_ref[...], w_ref[...])
  xj_all = _unpack_slots(xj_ref[...])
  xr = jnp.dot(xj_all, _expand_mat(16, 32),
               preferred_element_type=jnp.float32)
  m_all = _tree_contract(xr * w, 32)
  olo_ref[...] = _assemble16(m_all[:, 0:16])
  ohi_ref[...] = _assemble16(m_all[:, 16:32])


def _msg3_body(a_ref, xlo_ref, xhi_ref, w_ref, o_ref):
  w = _mlp_w(a_ref[...], w_ref[...])
  xj_all = jnp.concatenate(
      [_unpack_slots(xlo_ref[...]), _unpack_slots(xhi_ref[...])], axis=1)
  xr = jnp.dot(xj_all, _expand_mat(32, 8),
               preferred_element_type=jnp.float32)
  m_all = _tree_contract(xr * w, 8)
  padded = jnp.concatenate(
      [m_all, jnp.zeros((BE, 8), jnp.float32)], axis=1)
  o_ref[...] = _assemble16(padded)


def _pk_spec():
  return pl.BlockSpec((R256, 128), lambda i: (i, 0))


def _msg_call(body, nxj, kdim, nout):
  return pl.pallas_call(
      body,
      grid=(E // BE,),
      in_specs=[_pk_spec()] * (1 + nxj) + [
          pl.BlockSpec((17, kdim), lambda i: (0, 0))],
      out_specs=[_pk_spec()] * nout if nout > 1 else _pk_spec(),
      out_shape=[jax.ShapeDtypeStruct((E // G, 128), jnp.float32)] * nout
      if nout > 1 else jax.ShapeDtypeStruct((E // G, 128), jnp.float32),
  )


def _combine1_body(p_ref, x_ref, r_ref, b_ref, h_ref, cnt_ref):
  s = p_ref[0, 0] + p_ref[1, 0]
  cnt = p_ref[0, 1, :, 0:1] + p_ref[1, 1, :, 0:1]
  cntc = jnp.maximum(cnt, 1.0)
  root = jnp.dot(x_ref[...], r_ref[...], preferred_element_type=jnp.float32)
  h_ref[...] = jax.nn.relu(s / cntc + root + b_ref[...])
  cnt_ref[...] = cntc


def _combine2_body(p_ref, cnt_ref, h_ref, r_ref, b_ref, olo_ref, ohi_ref):
  s = jnp.concatenate([p_ref[0, 0] + p_ref[1, 0],
                       p_ref[0, 1] + p_ref[1, 1]], axis=1)
  root = jnp.dot(h_ref[...], r_ref[...], preferred_element_type=jnp.float32)
  h2 = jax.nn.relu(s / cnt_ref[...] + root + b_ref[...])
  olo_ref[...] = h2[:, 0:16]
  ohi_ref[...] = h2[:, 16:32]


def _combine3_body(p_ref, cnt_ref, hlo_ref, hhi_ref, r_ref, b_ref,
                   o_ref, ot_ref):
  s = p_ref[0, :, 0:8] + p_ref[1, :, 0:8]
  h2 = jnp.concatenate([hlo_ref[...], hhi_ref[...]], axis=1)
  root = jnp.dot(h2, r_ref[...], preferred_element_type=jnp.float32)
  h3 = jax.nn.relu(s / cnt_ref[...] + root + b_ref[...])
  o_ref[...] = h3
  ot_ref[...] = h3.T


def _cbt_body(hblk_ref, ht_ref, o_ref):
  ha = hblk_ref[...]
  ht = ht_ref[...]
  acc = jnp.abs(ht[0:1, :] - ha[:, 0:1])
  for k in range(1, 8):
    acc = acc + jnp.abs(ht[k:k + 1, :] - ha[:, k:k + 1])
  o_ref[...] = acc


TA = 256


def kernel(x, edge_index, edge_attr, lin1_W, lin1_b, root1, bias1,
           lin2_W, lin2_b, root2, bias2, lin3_W, lin3_b, root3, bias3):
  f32 = jnp.float32
  src = edge_index[0].reshape(NW, NCH, CH)
  dst = edge_index[1].reshape(NW, NCH, CH)
  x16 = jnp.pad(x, ((0, 0), (0, 15)))
  ap = edge_attr.reshape(E // G, 128)

  gather16 = _make_gather16()
  gather_pair = _make_gather_pair()
  scatter_pair = _make_scatter_pair()
  scatter16 = _make_scatter16()

  waug1 = jnp.concatenate([lin1_W, lin1_b[None, :]], axis=0)
  waug2 = jnp.concatenate([lin2_W, lin2_b[None, :]], axis=0)
  waug3 = jnp.concatenate([lin3_W, lin3_b[None, :]], axis=0)

  xj1 = gather16(x16, src)
  m1lo, m1hi = _msg_call(_msg1_body, 1, 16, 2)(
      ap, xj1.reshape(E // G, 128), waug1)
  p1 = scatter_pair(m1lo.reshape(E, 16), m1hi.reshape(E, 16), dst)
  h1, cntc = pl.pallas_call(
      _combine1_body,
      in_specs=[pl.BlockSpec((NC, 2, N, 16), lambda: (0, 0, 0, 0)),
                pl.BlockSpec((N, 1), lambda: (0, 0)),
                pl.BlockSpec((1, 16), lambda: (0, 0)),
                pl.BlockSpec((1, 16), lambda: (0, 0))],
      out_specs=[pl.BlockSpec((N, 16), lambda: (0, 0)),
                 pl.BlockSpec((N, 1), lambda: (0, 0))],
      out_shape=[jax.ShapeDtypeStruct((N, 16), f32),
                 jax.ShapeDtypeStruct((N, 1), f32)],
  )(p1, x, root1, bias1.reshape(1, 16))

  xj2 = gather16(h1, src)
  m2lo, m2hi = _msg_call(_msg2_body, 1, 16 * 32, 2)(
      ap, xj2.reshape(E // G, 128), waug2)
  p2 = scatter_pair(m2lo.reshape(E, 16), m2hi.reshape(E, 16), dst)
  h2lo, h2hi = pl.pallas_call(
      _combine2_body,
      in_specs=[pl.BlockSpec((NC, 2, N, 16), lambda: (0, 0, 0, 0)),
                pl.BlockSpec((N, 1), lambda: (0, 0)),
                pl.BlockSpec((N, 16), lambda: (0, 0)),
                pl.BlockSpec((16, 32), lambda: (0, 0)),
                pl.BlockSpec((1, 32), lambda: (0, 0))],
      out_specs=[pl.BlockSpec((N, 16), lambda: (0, 0)),
                 pl.BlockSpec((N, 16), lambda: (0, 0))],
      out_shape=[jax.ShapeDtypeStruct((N, 16), f32),
                 jax.ShapeDtypeStruct((N, 16), f32)],
  )(p2, cntc, h1, root2, bias2.reshape(1, 32))

  xj3lo, xj3hi = gather_pair(h2lo, h2hi, src)
  msg3 = _msg_call(_msg3_body, 2, 16 * 16, 1)(
      ap, xj3lo.reshape(E // G, 128), xj3hi.reshape(E // G, 128), waug3)
  p3 = scatter16(msg3.reshape(E, 16), dst)
  h3, h3t = pl.pallas_call(
      _combine3_body,
      in_specs=[pl.BlockSpec((NC, N, 16), lambda: (0, 0, 0)),
                pl.BlockSpec((N, 1), lambda: (0, 0)),
                pl.BlockSpec((N, 16), lambda: (0, 0)),
                pl.BlockSpec((N, 16), lambda: (0, 0)),
                pl.BlockSpec((32, 8), lambda: (0, 0)),
                pl.BlockSpec((1, 8), lambda: (0, 0))],
      out_specs=[pl.BlockSpec((N, 8), lambda: (0, 0)),
                 pl.BlockSpec((8, N), lambda: (0, 0))],
      out_shape=[jax.ShapeDtypeStruct((N, 8), f32),
                 jax.ShapeDtypeStruct((8, N), f32)],
  )(p3, cntc, h2lo, h2hi, root3, bias3.reshape(1, 8))

  cbt = pl.pallas_call(
      _cbt_body,
      grid=(N // TA,),
      in_specs=[pl.BlockSpec((TA, 8), lambda i: (i, 0)),
                pl.BlockSpec((8, N), lambda i: (0, 0))],
      out_specs=pl.BlockSpec((TA, N), lambda i: (i, 0)),
      out_shape=jax.ShapeDtypeStruct((N, N), f32),
  )(h3, h3t)
  return cbt


# speedup vs baseline: 5.3864x; 1.0042x over previous
import functools

import jax
import jax.numpy as jnp
from jax import lax
from jax.experimental import pallas as pl
from jax.experimental.pallas import tpu as pltpu
from jax.experimental.pallas import tpu_sc as plsc

N = 2048
E = 65536
NV = 16

NC = 2
NS = 16
NW = NC * NS
EPW = E // NW
CH = 128
NCH = EPW // CH
RPT = N // NS

BE = 2048
G = 8


@functools.lru_cache(maxsize=None)
def _mesh():
  return plsc.VectorSubcoreMesh(
      core_axis_name="c", subcore_axis_name="s", num_cores=NC, num_subcores=NS)


_SC_PARAMS = pltpu.CompilerParams(use_tc_tiling_on_sc=False)


def _gather_work(table_hbm, out_hbm, idx_v, rows_v, sem, wid):
  cps = []
  for ch in range(NCH):
    cps.append(pltpu.async_copy(
        table_hbm.at[idx_v.at[ch]], rows_v.at[pl.ds(ch * CH, CH)], sem))
  for cp in cps:
    cp.wait()
  pltpu.sync_copy(rows_v, out_hbm.at[pl.ds(wid * EPW, EPW)])


def _gather16_body(table_hbm, ei_hbm, out_hbm, idx_v, rows_v, sem):
  wid = lax.axis_index("s") * NC + lax.axis_index("c")
  pltpu.sync_copy(ei_hbm.at[0, wid], idx_v)
  _gather_work(table_hbm, out_hbm, idx_v, rows_v, sem, wid)


def _make_gather16():
  return pl.kernel(
      _gather16_body,
      out_type=jax.ShapeDtypeStruct((E, 16), jnp.float32),
      mesh=_mesh(),
      compiler_params=_SC_PARAMS,
      scratch_types=[
          pltpu.VMEM((NCH, CH), jnp.int32),
          pltpu.VMEM((EPW, 16), jnp.float32),
          pltpu.SemaphoreType.DMA,
      ])


def _gather_pair_body(tlo_hbm, thi_hbm, ei_hbm, olo_hbm, ohi_hbm,
                      idx_v, rlo_v, rhi_v, sem):
  wid = lax.axis_index("s") * NC + lax.axis_index("c")
  pltpu.sync_copy(ei_hbm.at[0, wid], idx_v)
  _gather_work(tlo_hbm, olo_hbm, idx_v, rlo_v, sem, wid)
  _gather_work(thi_hbm, ohi_hbm, idx_v, rhi_v, sem, wid)


def _make_gather_pair():
  out16 = jax.ShapeDtypeStruct((E, 16), jnp.float32)
  return pl.kernel(
      _gather_pair_body,
      out_type=[out16, out16],
      mesh=_mesh(),
      compiler_params=_SC_PARAMS,
      scratch_types=[
          pltpu.VMEM((NCH, CH), jnp.int32),
          pltpu.VMEM((EPW, 16), jnp.float32),
          pltpu.VMEM((EPW, 16), jnp.float32),
          pltpu.SemaphoreType.DMA,
      ])


def _zero_accum(zbuf, accum, sid):
  zv = jnp.zeros((16,), jnp.float32)
  for r in range(RPT):
    zbuf[r, pl.ds(0, 16)] = zv
  pltpu.sync_copy(zbuf, accum.at[pl.ds(sid * RPT, RPT)])


def _scatter_stream(msg_hbm, idx_v, msg_v, accum, wid, sem):
  pltpu.sync_copy(msg_hbm.at[pl.ds(wid * EPW, EPW)], msg_v)
  cps = []
  for ch in range(NCH):
    cps.append(pltpu.async_copy(
        msg_v.at[pl.ds(ch * CH, CH)], accum.at[idx_v.at[ch]], sem,
        add=True))
  for cp in cps:
    cp.wait()


def _scatter16_body(msg_hbm, ei_hbm, out_hbm, idx_v, msg_v, zbuf, accum,
                    sem):
  cid = lax.axis_index("c")
  sid = lax.axis_index("s")
  wid = sid * NC + cid
  _zero_accum(zbuf, accum, sid)
  plsc.subcore_barrier()
  pltpu.sync_copy(ei_hbm.at[1, wid], idx_v)
  _scatter_stream(msg_hbm, idx_v, msg_v, accum, wid, sem)
  plsc.subcore_barrier()
  pltpu.sync_copy(accum.at[pl.ds(sid * RPT, RPT)],
                  out_hbm.at[cid, pl.ds(sid * RPT, RPT)])


def _make_scatter16():
  return pl.kernel(
      _scatter16_body,
      out_type=jax.ShapeDtypeStruct((NC, N, 16), jnp.float32),
      mesh=_mesh(),
      compiler_params=_SC_PARAMS,
      scratch_types=[
          pltpu.VMEM((NCH, CH), jnp.int32),
          pltpu.VMEM((EPW, 16), jnp.float32),
          pltpu.VMEM((RPT, 16), jnp.float32),
          pltpu.VMEM_SHARED((N, 16), jnp.float32),
          pltpu.SemaphoreType.DMA,
      ])


def _scatter_pair_body(mlo_hbm, mhi_hbm, ei_hbm, out_hbm,
                       idx_v, vlo, vhi, zbuf, alo, ahi, sem):
  cid = lax.axis_index("c")
  sid = lax.axis_index("s")
  wid = sid * NC + cid
  _zero_accum(zbuf, alo, sid)
  _zero_accum(zbuf, ahi, sid)
  plsc.subcore_barrier()
  pltpu.sync_copy(ei_hbm.at[1, wid], idx_v)
  _scatter_stream(mlo_hbm, idx_v, vlo, alo, wid, sem)
  _scatter_stream(mhi_hbm, idx_v, vhi, ahi, wid, sem)
  plsc.subcore_barrier()
  pltpu.sync_copy(alo.at[pl.ds(sid * RPT, RPT)],
                  out_hbm.at[cid, 0, pl.ds(sid * RPT, RPT)])
  pltpu.sync_copy(ahi.at[pl.ds(sid * RPT, RPT)],
                  out_hbm.at[cid, 1, pl.ds(sid * RPT, RPT)])


def _make_scatter_pair():
  return pl.kernel(
      _scatter_pair_body,
      out_type=jax.ShapeDtypeStruct((NC, 2, N, 16), jnp.float32),
      mesh=_mesh(),
      compiler_params=_SC_PARAMS,
      scratch_types=[
          pltpu.VMEM((NCH, CH), jnp.int32),
          pltpu.VMEM((EPW, 16), jnp.float32),
          pltpu.VMEM((EPW, 16), jnp.float32),
          pltpu.VMEM((RPT, 16), jnp.float32),
          pltpu.VMEM_SHARED((N, 16), jnp.float32),
          pltpu.VMEM_SHARED((N, 16), jnp.float32),
          pltpu.SemaphoreType.DMA,
      ])


R256 = BE // G


def _expand_mat(c_in, c_out):
  kdim = c_in * c_out
  ri = lax.broadcasted_iota(jnp.int32, (c_in, kdim), 0)
  rc = lax.broadcasted_iota(jnp.int32, (c_in, kdim), 1)
  return (rc // c_out == ri).astype(jnp.float32)


def _tree_contract(acc, c_out):
  width = acc.shape[1]
  while width > c_out:
    width //= 2
    acc = acc[:, :width] + acc[:, width:]
  return acc


def _unpack_slots(pk):
  return jnp.concatenate([pk[:, g * 16:(g + 1) * 16] for g in range(G)],
                         axis=0)


def _assemble16(m_all):
  return jnp.concatenate([m_all[g * R256:(g + 1) * R256] for g in range(G)],
                         axis=1)


def _mlp_w(a_pk, waug):
  a_all = _unpack_slots(a_pk)
  aug = jnp.concatenate([a_all, jnp.ones((BE, 1), jnp.float32)], axis=1)
  return jax.nn.relu(
      jnp.dot(aug, waug, preferred_element_type=jnp.float32))


def _msg1_body(a_ref, xj_ref, w_ref, olo_ref, ohi_ref):
  w = _mlp_w(a_ref[...], w_ref[...])
  xj_all = _unpack_slots(xj_ref[...])
  m = xj_all[:, 0:1] * w
  olo_ref[...] = _assemble16(m)
  lanes = lax.broadcasted_iota(jnp.int32, (R256, 128), 1)
  ohi_ref[...] = (lanes % 16 == 0).astype(jnp.float32)


def _msg2_body(a_ref, xj_ref, w_ref, olo_ref, ohi_ref):
  w = _mlp_w(a_ref[...], w_ref[...])
  xj_all = _unpack_slots(xj_ref[...])
  xr = jnp.dot(xj_all, _expand_mat(16, 32),
               preferred_element_type=jnp.float32)
  m_all = _tree_contract(xr * w, 32)
  olo_ref[...] = _assemble16(m_all[:, 0:16])
  ohi_ref[...] = _assemble16(m_all[:, 16:32])


def _msg3_body(a_ref, xlo_ref, xhi_ref, w_ref, o_ref):
  w = _mlp_w(a_ref[...], w_ref[...])
  xj_all = jnp.concatenate(
      [_unpack_slots(xlo_ref[...]), _unpack_slots(xhi_ref[...])], axis=1)
  xr = jnp.dot(xj_all, _expand_mat(32, 8),
               preferred_element_type=jnp.float32)
  m_all = _tree_contract(xr * w, 8)
  padded = jnp.concatenate(
      [m_all, jnp.zeros((BE, 8), jnp.float32)], axis=1)
  o_ref[...] = _assemble16(padded)


def _pk_spec():
  return pl.BlockSpec((R256, 128), lambda i: (i, 0))


def _msg_call(body, nxj, kdim, nout):
  return pl.pallas_call(
      body,
      grid=(E // BE,),
      in_specs=[_pk_spec()] * (1 + nxj) + [
          pl.BlockSpec((17, kdim), lambda i: (0, 0))],
      out_specs=[_pk_spec()] * nout if nout > 1 else _pk_spec(),
      out_shape=[jax.ShapeDtypeStruct((E // G, 128), jnp.float32)] * nout
      if nout > 1 else jax.ShapeDtypeStruct((E // G, 128), jnp.float32),
  )


def _combine1_body(p_ref, x_ref, r_ref, b_ref, h_ref, cnt_ref):
  s = p_ref[0, 0] + p_ref[1, 0]
  cnt = p_ref[0, 1, :, 0:1] + p_ref[1, 1, :, 0:1]
  cntc = jnp.maximum(cnt, 1.0)
  root = jnp.dot(x_ref[...], r_ref[...], preferred_element_type=jnp.float32)
  h_ref[...] = jax.nn.relu(s / cntc + root + b_ref[...])
  cnt_ref[...] = cntc


def _combine2_body(p_ref, cnt_ref, h_ref, r_ref, b_ref, olo_ref, ohi_ref):
  s = jnp.concatenate([p_ref[0, 0] + p_ref[1, 0],
                       p_ref[0, 1] + p_ref[1, 1]], axis=1)
  root = jnp.dot(h_ref[...], r_ref[...], preferred_element_type=jnp.float32)
  h2 = jax.nn.relu(s / cnt_ref[...] + root + b_ref[...])
  olo_ref[...] = h2[:, 0:16]
  ohi_ref[...] = h2[:, 16:32]


def _combine3_body(p_ref, cnt_ref, hlo_ref, hhi_ref, r_ref, b_ref,
                   o_ref, ot_ref):
  s = p_ref[0, :, 0:8] + p_ref[1, :, 0:8]
  h2 = jnp.concatenate([hlo_ref[...], hhi_ref[...]], axis=1)
  root = jnp.dot(h2, r_ref[...], preferred_element_type=jnp.float32)
  h3 = jax.nn.relu(s / cnt_ref[...] + root + b_ref[...])
  o_ref[...] = h3
  ot_ref[...] = h3.T


def _cbt_body(hblk_ref, ht_ref, o_ref):
  ha = hblk_ref[...]
  ht = ht_ref[...]
  acc = jnp.abs(ht[0:1, :] - ha[:, 0:1])
  for k in range(1, 8):
    acc = acc + jnp.abs(ht[k:k + 1, :] - ha[:, k:k + 1])
  o_ref[...] = acc


TA = 256


def kernel(x, edge_index, edge_attr, lin1_W, lin1_b, root1, bias1,
           lin2_W, lin2_b, root2, bias2, lin3_W, lin3_b, root3, bias3):
  f32 = jnp.float32
  ei = edge_index.reshape(2, NW, NCH, CH)
  x16 = jnp.pad(x, ((0, 0), (0, 15)))
  ap = edge_attr.reshape(E // G, 128)

  gather16 = _make_gather16()
  gather_pair = _make_gather_pair()
  scatter_pair = _make_scatter_pair()
  scatter16 = _make_scatter16()

  waug1 = jnp.concatenate([lin1_W, lin1_b[None, :]], axis=0)
  waug2 = jnp.concatenate([lin2_W, lin2_b[None, :]], axis=0)
  waug3 = jnp.concatenate([lin3_W, lin3_b[None, :]], axis=0)

  xj1 = gather16(x16, ei)
  m1lo, m1hi = _msg_call(_msg1_body, 1, 16, 2)(
      ap, xj1.reshape(E // G, 128), waug1)
  p1 = scatter_pair(m1lo.reshape(E, 16), m1hi.reshape(E, 16), ei)
  h1, cntc = pl.pallas_call(
      _combine1_body,
      in_specs=[pl.BlockSpec((NC, 2, N, 16), lambda: (0, 0, 0, 0)),
                pl.BlockSpec((N, 1), lambda: (0, 0)),
                pl.BlockSpec((1, 16), lambda: (0, 0)),
                pl.BlockSpec((1, 16), lambda: (0, 0))],
      out_specs=[pl.BlockSpec((N, 16), lambda: (0, 0)),
                 pl.BlockSpec((N, 1), lambda: (0, 0))],
      out_shape=[jax.ShapeDtypeStruct((N, 16), f32),
                 jax.ShapeDtypeStruct((N, 1), f32)],
  )(p1, x, root1, bias1.reshape(1, 16))

  xj2 = gather16(h1, ei)
  m2lo, m2hi = _msg_call(_msg2_body, 1, 16 * 32, 2)(
      ap, xj2.reshape(E // G, 128), waug2)
  p2 = scatter_pair(m2lo.reshape(E, 16), m2hi.reshape(E, 16), ei)
  h2lo, h2hi = pl.pallas_call(
      _combine2_body,
      in_specs=[pl.BlockSpec((NC, 2, N, 16), lambda: (0, 0, 0, 0)),
                pl.BlockSpec((N, 1), lambda: (0, 0)),
                pl.BlockSpec((N, 16), lambda: (0, 0)),
                pl.BlockSpec((16, 32), lambda: (0, 0)),
                pl.BlockSpec((1, 32), lambda: (0, 0))],
      out_specs=[pl.BlockSpec((N, 16), lambda: (0, 0)),
                 pl.BlockSpec((N, 16), lambda: (0, 0))],
      out_shape=[jax.ShapeDtypeStruct((N, 16), f32),
                 jax.ShapeDtypeStruct((N, 16), f32)],
  )(p2, cntc, h1, root2, bias2.reshape(1, 32))

  xj3lo, xj3hi = gather_pair(h2lo, h2hi, ei)
  msg3 = _msg_call(_msg3_body, 2, 16 * 16, 1)(
      ap, xj3lo.reshape(E // G, 128), xj3hi.reshape(E // G, 128), waug3)
  p3 = scatter16(msg3.reshape(E, 16), ei)
  h3, h3t = pl.pallas_call(
      _combine3_body,
      in_specs=[pl.BlockSpec((NC, N, 16), lambda: (0, 0, 0)),
                pl.BlockSpec((N, 1), lambda: (0, 0)),
                pl.BlockSpec((N, 16), lambda: (0, 0)),
                pl.BlockSpec((N, 16), lambda: (0, 0)),
                pl.BlockSpec((32, 8), lambda: (0, 0)),
                pl.BlockSpec((1, 8), lambda: (0, 0))],
      out_specs=[pl.BlockSpec((N, 8), lambda: (0, 0)),
                 pl.BlockSpec((8, N), lambda: (0, 0))],
      out_shape=[jax.ShapeDtypeStruct((N, 8), f32),
                 jax.ShapeDtypeStruct((8, N), f32)],
  )(p3, cntc, h2lo, h2hi, root3, bias3.reshape(1, 8))

  cbt = pl.pallas_call(
      _cbt_body,
      grid=(N // TA,),
      in_specs=[pl.BlockSpec((TA, 8), lambda i: (i, 0)),
                pl.BlockSpec((8, N), lambda i: (0, 0))],
      out_specs=pl.BlockSpec((TA, N), lambda i: (i, 0)),
      out_shape=jax.ShapeDtypeStruct((N, N), f32),
  )(h3, h3t)
  return cbt


# speedup vs baseline: 5.8484x; 1.0858x over previous
import functools

import jax
import jax.numpy as jnp
from jax import lax
from jax.experimental import pallas as pl
from jax.experimental.pallas import tpu as pltpu
from jax.experimental.pallas import tpu_sc as plsc

N = 2048
E = 65536
NV = 16

NC = 2
NS = 16
NW = NC * NS
EPW = E // NW
CH = 128
NCH = EPW // CH
RPT = N // NS

BE = 2048
G = 8


@functools.lru_cache(maxsize=None)
def _mesh():
  return plsc.VectorSubcoreMesh(
      core_axis_name="c", subcore_axis_name="s", num_cores=NC, num_subcores=NS)


_SC_PARAMS = pltpu.CompilerParams(use_tc_tiling_on_sc=False)


def _gather_work(table_hbm, out_hbm, idx_v, rows_v, sem, wid):
  cps = []
  for ch in range(NCH):
    cps.append(pltpu.async_copy(
        table_hbm.at[idx_v.at[ch]], rows_v.at[pl.ds(ch * CH, CH)], sem))
  for cp in cps:
    cp.wait()
  pltpu.sync_copy(rows_v, out_hbm.at[pl.ds(wid * EPW, EPW)])


def _gather16_body(table_hbm, ei_hbm, out_hbm, idx_v, rows_v, sem):
  wid = lax.axis_index("s") * NC + lax.axis_index("c")
  pltpu.sync_copy(ei_hbm.at[0, wid], idx_v)
  _gather_work(table_hbm, out_hbm, idx_v, rows_v, sem, wid)


def _make_gather16():
  return pl.kernel(
      _gather16_body,
      out_type=jax.ShapeDtypeStruct((E, 16), jnp.float32),
      mesh=_mesh(),
      compiler_params=_SC_PARAMS,
      scratch_types=[
          pltpu.VMEM((NCH, CH), jnp.int32),
          pltpu.VMEM((EPW, 16), jnp.float32),
          pltpu.SemaphoreType.DMA,
      ])


def _gather_pair_body(tlo_hbm, thi_hbm, ei_hbm, olo_hbm, ohi_hbm,
                      idx_v, rlo_v, rhi_v, sem):
  wid = lax.axis_index("s") * NC + lax.axis_index("c")
  pltpu.sync_copy(ei_hbm.at[0, wid], idx_v)
  _gather_work(tlo_hbm, olo_hbm, idx_v, rlo_v, sem, wid)
  _gather_work(thi_hbm, ohi_hbm, idx_v, rhi_v, sem, wid)


def _make_gather_pair():
  out16 = jax.ShapeDtypeStruct((E, 16), jnp.float32)
  return pl.kernel(
      _gather_pair_body,
      out_type=[out16, out16],
      mesh=_mesh(),
      compiler_params=_SC_PARAMS,
      scratch_types=[
          pltpu.VMEM((NCH, CH), jnp.int32),
          pltpu.VMEM((EPW, 16), jnp.float32),
          pltpu.VMEM((EPW, 16), jnp.float32),
          pltpu.SemaphoreType.DMA,
      ])


def _zero_accum(zbuf, accum, sid):
  zv = jnp.zeros((16,), jnp.float32)
  for r in range(RPT):
    zbuf[r, pl.ds(0, 16)] = zv
  pltpu.sync_copy(zbuf, accum.at[pl.ds(sid * RPT, RPT)])


def _scatter_stream(msg_hbm, idx_v, msg_v, accum, wid, sem):
  pltpu.sync_copy(msg_hbm.at[pl.ds(wid * EPW, EPW)], msg_v)
  cps = []
  for ch in range(NCH):
    cps.append(pltpu.async_copy(
        msg_v.at[pl.ds(ch * CH, CH)], accum.at[idx_v.at[ch]], sem,
        add=True))
  for cp in cps:
    cp.wait()


def _scatter16_body(msg_hbm, ei_hbm, out_hbm, idx_v, msg_v, zbuf, accum,
                    sem):
  cid = lax.axis_index("c")
  sid = lax.axis_index("s")
  wid = sid * NC + cid
  _zero_accum(zbuf, accum, sid)
  plsc.subcore_barrier()
  pltpu.sync_copy(ei_hbm.at[1, wid], idx_v)
  _scatter_stream(msg_hbm, idx_v, msg_v, accum, wid, sem)
  plsc.subcore_barrier()
  pltpu.sync_copy(accum.at[pl.ds(sid * RPT, RPT)],
                  out_hbm.at[cid, pl.ds(sid * RPT, RPT)])


def _make_scatter16():
  return pl.kernel(
      _scatter16_body,
      out_type=jax.ShapeDtypeStruct((NC, N, 16), jnp.float32),
      mesh=_mesh(),
      compiler_params=_SC_PARAMS,
      scratch_types=[
          pltpu.VMEM((NCH, CH), jnp.int32),
          pltpu.VMEM((EPW, 16), jnp.float32),
          pltpu.VMEM((RPT, 16), jnp.float32),
          pltpu.VMEM_SHARED((N, 16), jnp.float32),
          pltpu.SemaphoreType.DMA,
      ])


def _scatter_pair_body(mlo_hbm, mhi_hbm, ei_hbm, out_hbm,
                       idx_v, vlo, vhi, zbuf, alo, ahi, sem):
  cid = lax.axis_index("c")
  sid = lax.axis_index("s")
  wid = sid * NC + cid
  _zero_accum(zbuf, alo, sid)
  _zero_accum(zbuf, ahi, sid)
  plsc.subcore_barrier()
  pltpu.sync_copy(ei_hbm.at[1, wid], idx_v)
  _scatter_stream(mlo_hbm, idx_v, vlo, alo, wid, sem)
  _scatter_stream(mhi_hbm, idx_v, vhi, ahi, wid, sem)
  plsc.subcore_barrier()
  pltpu.sync_copy(alo.at[pl.ds(sid * RPT, RPT)],
                  out_hbm.at[cid, 0, pl.ds(sid * RPT, RPT)])
  pltpu.sync_copy(ahi.at[pl.ds(sid * RPT, RPT)],
                  out_hbm.at[cid, 1, pl.ds(sid * RPT, RPT)])


def _make_scatter_pair():
  return pl.kernel(
      _scatter_pair_body,
      out_type=jax.ShapeDtypeStruct((NC, 2, N, 16), jnp.float32),
      mesh=_mesh(),
      compiler_params=_SC_PARAMS,
      scratch_types=[
          pltpu.VMEM((NCH, CH), jnp.int32),
          pltpu.VMEM((EPW, 16), jnp.float32),
          pltpu.VMEM((EPW, 16), jnp.float32),
          pltpu.VMEM((RPT, 16), jnp.float32),
          pltpu.VMEM_SHARED((N, 16), jnp.float32),
          pltpu.VMEM_SHARED((N, 16), jnp.float32),
          pltpu.SemaphoreType.DMA,
      ])


R256 = BE // G


def _expand_mat(c_in, c_out):
  kdim = c_in * c_out
  ri = lax.broadcasted_iota(jnp.int32, (c_in, kdim), 0)
  rc = lax.broadcasted_iota(jnp.int32, (c_in, kdim), 1)
  return (rc // c_out == ri).astype(jnp.float32)


def _tree_contract(acc, c_out):
  width = acc.shape[1]
  while width > c_out:
    width //= 2
    acc = acc[:, :width] + acc[:, width:]
  return acc


def _unpack_slots(pk):
  return jnp.concatenate([pk[:, g * 16:(g + 1) * 16] for g in range(G)],
                         axis=0)


def _assemble16(m_all):
  return jnp.concatenate([m_all[g * R256:(g + 1) * R256] for g in range(G)],
                         axis=1)


def _mlp_w(a_pk, waug):
  a_all = _unpack_slots(a_pk)
  aug = jnp.concatenate([a_all, jnp.ones((BE, 1), jnp.float32)], axis=1)
  return jax.nn.relu(
      jnp.dot(aug, waug, preferred_element_type=jnp.float32))


def _msg1_body(a_ref, xj_ref, w_ref, olo_ref, ohi_ref):
  w = _mlp_w(a_ref[...], w_ref[...])
  xj_all = _unpack_slots(xj_ref[...])
  m = xj_all[:, 0:1] * w
  olo_ref[...] = _assemble16(m)
  lanes = lax.broadcasted_iota(jnp.int32, (R256, 128), 1)
  ohi_ref[...] = (lanes % 16 == 0).astype(jnp.float32)


def _contract_mx(acc, c_in, c_out):
  kdim = c_in * c_out
  si = lax.broadcasted_iota(jnp.int32, (kdim, c_out), 0)
  so = lax.broadcasted_iota(jnp.int32, (kdim, c_out), 1)
  sm = (si % c_out == so).astype(jnp.bfloat16)
  return jnp.dot(acc.astype(jnp.bfloat16), sm,
                 preferred_element_type=jnp.float32)


def _msg2_body(a_ref, xj_ref, w_ref, olo_ref, ohi_ref):
  w = _mlp_w(a_ref[...], w_ref[...])
  xj_all = _unpack_slots(xj_ref[...])
  xr = jnp.dot(xj_all, _expand_mat(16, 32),
               preferred_element_type=jnp.float32)
  m_all = _contract_mx(xr * w, 16, 32)
  olo_ref[...] = _assemble16(m_all[:, 0:16])
  ohi_ref[...] = _assemble16(m_all[:, 16:32])


def _msg3_body(a_ref, xlo_ref, xhi_ref, w_ref, o_ref):
  w = _mlp_w(a_ref[...], w_ref[...])
  xj_all = jnp.concatenate(
      [_unpack_slots(xlo_ref[...]), _unpack_slots(xhi_ref[...])], axis=1)
  xr = jnp.dot(xj_all, _expand_mat(32, 8),
               preferred_element_type=jnp.float32)
  m_all = _contract_mx(xr * w, 32, 8)
  padded = jnp.concatenate(
      [m_all, jnp.zeros((BE, 8), jnp.float32)], axis=1)
  o_ref[...] = _assemble16(padded)


def _pk_spec():
  return pl.BlockSpec((R256, 128), lambda i: (i, 0))


def _msg_call(body, nxj, kdim, nout):
  return pl.pallas_call(
      body,
      grid=(E // BE,),
      in_specs=[_pk_spec()] * (1 + nxj) + [
          pl.BlockSpec((17, kdim), lambda i: (0, 0))],
      out_specs=[_pk_spec()] * nout if nout > 1 else _pk_spec(),
      out_shape=[jax.ShapeDtypeStruct((E // G, 128), jnp.float32)] * nout
      if nout > 1 else jax.ShapeDtypeStruct((E // G, 128), jnp.float32),
  )


def _combine1_body(p_ref, x_ref, r_ref, b_ref, h_ref, cnt_ref):
  s = p_ref[0, 0] + p_ref[1, 0]
  cnt = p_ref[0, 1, :, 0:1] + p_ref[1, 1, :, 0:1]
  cntc = jnp.maximum(cnt, 1.0)
  root = jnp.dot(x_ref[...], r_ref[...], preferred_element_type=jnp.float32)
  h_ref[...] = jax.nn.relu(s / cntc + root + b_ref[...])
  cnt_ref[...] = cntc


def _combine2_body(p_ref, cnt_ref, h_ref, r_ref, b_ref, olo_ref, ohi_ref):
  s = jnp.concatenate([p_ref[0, 0] + p_ref[1, 0],
                       p_ref[0, 1] + p_ref[1, 1]], axis=1)
  root = jnp.dot(h_ref[...], r_ref[...], preferred_element_type=jnp.float32)
  h2 = jax.nn.relu(s / cnt_ref[...] + root + b_ref[...])
  olo_ref[...] = h2[:, 0:16]
  ohi_ref[...] = h2[:, 16:32]


def _combine3_body(p_ref, cnt_ref, hlo_ref, hhi_ref, r_ref, b_ref,
                   o_ref, ot_ref):
  s = p_ref[0, :, 0:8] + p_ref[1, :, 0:8]
  h2 = jnp.concatenate([hlo_ref[...], hhi_ref[...]], axis=1)
  root = jnp.dot(h2, r_ref[...], preferred_element_type=jnp.float32)
  h3 = jax.nn.relu(s / cnt_ref[...] + root + b_ref[...])
  o_ref[...] = h3
  ot_ref[...] = h3.T


def _cbt_body(hblk_ref, ht_ref, o_ref):
  ha = hblk_ref[...]
  ht = ht_ref[...]
  acc = jnp.abs(ht[0:1, :] - ha[:, 0:1])
  for k in range(1, 8):
    acc = acc + jnp.abs(ht[k:k + 1, :] - ha[:, k:k + 1])
  o_ref[...] = acc


TA = 256


def kernel(x, edge_index, edge_attr, lin1_W, lin1_b, root1, bias1,
           lin2_W, lin2_b, root2, bias2, lin3_W, lin3_b, root3, bias3):
  f32 = jnp.float32
  ei = edge_index.reshape(2, NW, NCH, CH)
  x16 = jnp.pad(x, ((0, 0), (0, 15)))
  ap = edge_attr.reshape(E // G, 128)

  gather16 = _make_gather16()
  gather_pair = _make_gather_pair()
  scatter_pair = _make_scatter_pair()
  scatter16 = _make_scatter16()

  waug1 = jnp.concatenate([lin1_W, lin1_b[None, :]], axis=0)
  waug2 = jnp.concatenate([lin2_W, lin2_b[None, :]], axis=0)
  waug3 = jnp.concatenate([lin3_W, lin3_b[None, :]], axis=0)

  xj1 = gather16(x16, ei)
  m1lo, m1hi = _msg_call(_msg1_body, 1, 16, 2)(
      ap, xj1.reshape(E // G, 128), waug1)
  p1 = scatter_pair(m1lo.reshape(E, 16), m1hi.reshape(E, 16), ei)
  h1, cntc = pl.pallas_call(
      _combine1_body,
      in_specs=[pl.BlockSpec((NC, 2, N, 16), lambda: (0, 0, 0, 0)),
                pl.BlockSpec((N, 1), lambda: (0, 0)),
                pl.BlockSpec((1, 16), lambda: (0, 0)),
                pl.BlockSpec((1, 16), lambda: (0, 0))],
      out_specs=[pl.BlockSpec((N, 16), lambda: (0, 0)),
                 pl.BlockSpec((N, 1), lambda: (0, 0))],
      out_shape=[jax.ShapeDtypeStruct((N, 16), f32),
                 jax.ShapeDtypeStruct((N, 1), f32)],
  )(p1, x, root1, bias1.reshape(1, 16))

  xj2 = gather16(h1, ei)
  m2lo, m2hi = _msg_call(_msg2_body, 1, 16 * 32, 2)(
      ap, xj2.reshape(E // G, 128), waug2)
  p2 = scatter_pair(m2lo.reshape(E, 16), m2hi.reshape(E, 16), ei)
  h2lo, h2hi = pl.pallas_call(
      _combine2_body,
      in_specs=[pl.BlockSpec((NC, 2, N, 16), lambda: (0, 0, 0, 0)),
                pl.BlockSpec((N, 1), lambda: (0, 0)),
                pl.BlockSpec((N, 16), lambda: (0, 0)),
                pl.BlockSpec((16, 32), lambda: (0, 0)),
                pl.BlockSpec((1, 32), lambda: (0, 0))],
      out_specs=[pl.BlockSpec((N, 16), lambda: (0, 0)),
                 pl.BlockSpec((N, 16), lambda: (0, 0))],
      out_shape=[jax.ShapeDtypeStruct((N, 16), f32),
                 jax.ShapeDtypeStruct((N, 16), f32)],
  )(p2, cntc, h1, root2, bias2.reshape(1, 32))

  xj3lo, xj3hi = gather_pair(h2lo, h2hi, ei)
  msg3 = _msg_call(_msg3_body, 2, 16 * 16, 1)(
      ap, xj3lo.reshape(E // G, 128), xj3hi.reshape(E // G, 128), waug3)
  p3 = scatter16(msg3.reshape(E, 16), ei)
  h3, h3t = pl.pallas_call(
      _combine3_body,
      in_specs=[pl.BlockSpec((NC, N, 16), lambda: (0, 0, 0)),
                pl.BlockSpec((N, 1), lambda: (0, 0)),
                pl.BlockSpec((N, 16), lambda: (0, 0)),
                pl.BlockSpec((N, 16), lambda: (0, 0)),
                pl.BlockSpec((32, 8), lambda: (0, 0)),
                pl.BlockSpec((1, 8), lambda: (0, 0))],
      out_specs=[pl.BlockSpec((N, 8), lambda: (0, 0)),
                 pl.BlockSpec((8, N), lambda: (0, 0))],
      out_shape=[jax.ShapeDtypeStruct((N, 8), f32),
                 jax.ShapeDtypeStruct((8, N), f32)],
  )(p3, cntc, h2lo, h2hi, root3, bias3.reshape(1, 8))

  cbt = pl.pallas_call(
      _cbt_body,
      grid=(N // TA,),
      in_specs=[pl.BlockSpec((TA, 8), lambda i: (i, 0)),
                pl.BlockSpec((8, N), lambda i: (0, 0))],
      out_specs=pl.BlockSpec((TA, N), lambda i: (i, 0)),
      out_shape=jax.ShapeDtypeStruct((N, N), f32),
  )(h3, h3t)
  return cbt


# speedup vs baseline: 6.0701x; 1.0379x over previous
import functools

import jax
import jax.numpy as jnp
from jax import lax
from jax.experimental import pallas as pl
from jax.experimental.pallas import tpu as pltpu
from jax.experimental.pallas import tpu_sc as plsc

N = 2048
E = 65536
NV = 16

NC = 2
NS = 16
NW = NC * NS
EPW = E // NW
CH = 128
NCH = EPW // CH
RPT = N // NS

BE = 2048
G = 8


@functools.lru_cache(maxsize=None)
def _mesh():
  return plsc.VectorSubcoreMesh(
      core_axis_name="c", subcore_axis_name="s", num_cores=NC, num_subcores=NS)


_SC_PARAMS = pltpu.CompilerParams(use_tc_tiling_on_sc=False)


def _gather_work(table_hbm, out_hbm, idx_v, rows_v, sem, wid):
  cps = []
  for ch in range(NCH):
    cps.append(pltpu.async_copy(
        table_hbm.at[idx_v.at[ch]], rows_v.at[pl.ds(ch * CH, CH)], sem))
  for cp in cps:
    cp.wait()
  pltpu.sync_copy(rows_v, out_hbm.at[pl.ds(wid * EPW, EPW)])


def _gather16_body(table_hbm, ei_hbm, out_hbm, idx_v, rows_v, sem):
  wid = lax.axis_index("s") * NC + lax.axis_index("c")
  pltpu.sync_copy(ei_hbm.at[0, wid], idx_v)
  _gather_work(table_hbm, out_hbm, idx_v, rows_v, sem, wid)


def _make_gather16():
  return pl.kernel(
      _gather16_body,
      out_type=jax.ShapeDtypeStruct((E, 16), jnp.float32),
      mesh=_mesh(),
      compiler_params=_SC_PARAMS,
      scratch_types=[
          pltpu.VMEM((NCH, CH), jnp.int32),
          pltpu.VMEM((EPW, 16), jnp.float32),
          pltpu.SemaphoreType.DMA,
      ])


def _gather_pair_body(tlo_hbm, thi_hbm, ei_hbm, olo_hbm, ohi_hbm,
                      idx_v, rlo_v, rhi_v, sem):
  wid = lax.axis_index("s") * NC + lax.axis_index("c")
  pltpu.sync_copy(ei_hbm.at[0, wid], idx_v)
  _gather_work(tlo_hbm, olo_hbm, idx_v, rlo_v, sem, wid)
  _gather_work(thi_hbm, ohi_hbm, idx_v, rhi_v, sem, wid)


def _make_gather_pair():
  out16 = jax.ShapeDtypeStruct((E, 16), jnp.float32)
  return pl.kernel(
      _gather_pair_body,
      out_type=[out16, out16],
      mesh=_mesh(),
      compiler_params=_SC_PARAMS,
      scratch_types=[
          pltpu.VMEM((NCH, CH), jnp.int32),
          pltpu.VMEM((EPW, 16), jnp.float32),
          pltpu.VMEM((EPW, 16), jnp.float32),
          pltpu.SemaphoreType.DMA,
      ])


def _zero_accum(zbuf, accum, sid):
  zv = jnp.zeros((16,), jnp.float32)
  for r in range(RPT):
    zbuf[r, pl.ds(0, 16)] = zv
  pltpu.sync_copy(zbuf, accum.at[pl.ds(sid * RPT, RPT)])


def _scatter_stream(msg_hbm, idx_v, msg_v, accum, wid, sem):
  pltpu.sync_copy(msg_hbm.at[pl.ds(wid * EPW, EPW)], msg_v)
  cps = []
  for ch in range(NCH):
    cps.append(pltpu.async_copy(
        msg_v.at[pl.ds(ch * CH, CH)], accum.at[idx_v.at[ch]], sem,
        add=True))
  for cp in cps:
    cp.wait()


def _scatter16_body(msg_hbm, ei_hbm, out_hbm, idx_v, msg_v, zbuf, accum,
                    sem):
  cid = lax.axis_index("c")
  sid = lax.axis_index("s")
  wid = sid * NC + cid
  _zero_accum(zbuf, accum, sid)
  plsc.subcore_barrier()
  pltpu.sync_copy(ei_hbm.at[1, wid], idx_v)
  _scatter_stream(msg_hbm, idx_v, msg_v, accum, wid, sem)
  plsc.subcore_barrier()
  pltpu.sync_copy(accum.at[pl.ds(sid * RPT, RPT)],
                  out_hbm.at[cid, pl.ds(sid * RPT, RPT)])


def _make_scatter16():
  return pl.kernel(
      _scatter16_body,
      out_type=jax.ShapeDtypeStruct((NC, N, 16), jnp.float32),
      mesh=_mesh(),
      compiler_params=_SC_PARAMS,
      scratch_types=[
          pltpu.VMEM((NCH, CH), jnp.int32),
          pltpu.VMEM((EPW, 16), jnp.float32),
          pltpu.VMEM((RPT, 16), jnp.float32),
          pltpu.VMEM_SHARED((N, 16), jnp.float32),
          pltpu.SemaphoreType.DMA,
      ])


def _scatter_pair_body(mlo_hbm, mhi_hbm, ei_hbm, out_hbm,
                       idx_v, vlo, vhi, zbuf, alo, ahi, sem):
  cid = lax.axis_index("c")
  sid = lax.axis_index("s")
  wid = sid * NC + cid
  _zero_accum(zbuf, alo, sid)
  _zero_accum(zbuf, ahi, sid)
  plsc.subcore_barrier()
  pltpu.sync_copy(ei_hbm.at[1, wid], idx_v)
  _scatter_stream(mlo_hbm, idx_v, vlo, alo, wid, sem)
  _scatter_stream(mhi_hbm, idx_v, vhi, ahi, wid, sem)
  plsc.subcore_barrier()
  pltpu.sync_copy(alo.at[pl.ds(sid * RPT, RPT)],
                  out_hbm.at[cid, 0, pl.ds(sid * RPT, RPT)])
  pltpu.sync_copy(ahi.at[pl.ds(sid * RPT, RPT)],
                  out_hbm.at[cid, 1, pl.ds(sid * RPT, RPT)])


def _make_scatter_pair():
  return pl.kernel(
      _scatter_pair_body,
      out_type=jax.ShapeDtypeStruct((NC, 2, N, 16), jnp.float32),
      mesh=_mesh(),
      compiler_params=_SC_PARAMS,
      scratch_types=[
          pltpu.VMEM((NCH, CH), jnp.int32),
          pltpu.VMEM((EPW, 16), jnp.float32),
          pltpu.VMEM((EPW, 16), jnp.float32),
          pltpu.VMEM((RPT, 16), jnp.float32),
          pltpu.VMEM_SHARED((N, 16), jnp.float32),
          pltpu.VMEM_SHARED((N, 16), jnp.float32),
          pltpu.SemaphoreType.DMA,
      ])


R256 = BE // G


def _expand_mat(c_in, c_out):
  kdim = c_in * c_out
  ri = lax.broadcasted_iota(jnp.int32, (c_in, kdim), 0)
  rc = lax.broadcasted_iota(jnp.int32, (c_in, kdim), 1)
  return (rc // c_out == ri).astype(jnp.float32)


def _tree_contract(acc, c_out):
  width = acc.shape[1]
  while width > c_out:
    width //= 2
    acc = acc[:, :width] + acc[:, width:]
  return acc


def _unpack_slots(pk):
  return jnp.concatenate([pk[:, g * 16:(g + 1) * 16] for g in range(G)],
                         axis=0)


def _assemble16(m_all):
  return jnp.concatenate([m_all[g * R256:(g + 1) * R256] for g in range(G)],
                         axis=1)


def _mlp_w(a_pk, waug):
  a_all = _unpack_slots(a_pk)
  aug = jnp.concatenate([a_all, jnp.ones((BE, 1), jnp.float32)], axis=1)
  return jax.nn.relu(
      jnp.dot(aug, waug, preferred_element_type=jnp.float32))


def _msg1_body(a_ref, xj_ref, w_ref, olo_ref, ohi_ref):
  w = _mlp_w(a_ref[...], w_ref[...])
  olo_ref[...] = xj_ref[...] * _assemble16(w)
  lanes = lax.broadcasted_iota(jnp.int32, (R256, 128), 1)
  ohi_ref[...] = (lanes % 16 == 0).astype(jnp.float32)


def _contract_mx(acc, c_in, c_out):
  kdim = c_in * c_out
  si = lax.broadcasted_iota(jnp.int32, (kdim, c_out), 0)
  so = lax.broadcasted_iota(jnp.int32, (kdim, c_out), 1)
  sm = (si % c_out == so).astype(jnp.bfloat16)
  return jnp.dot(acc.astype(jnp.bfloat16), sm,
                 preferred_element_type=jnp.float32)


def _msg2_body(a_ref, xj_ref, w_ref, olo_ref, ohi_ref):
  w = _mlp_w(a_ref[...], w_ref[...])
  xj_all = _unpack_slots(xj_ref[...])
  xr = jnp.dot(xj_all, _expand_mat(16, 32),
               preferred_element_type=jnp.float32)
  m_all = _contract_mx(xr * w, 16, 32)
  olo_ref[...] = _assemble16(m_all[:, 0:16])
  ohi_ref[...] = _assemble16(m_all[:, 16:32])


def _msg3_body(a_ref, xlo_ref, xhi_ref, w_ref, o_ref):
  w = _mlp_w(a_ref[...], w_ref[...])
  xj_all = jnp.concatenate(
      [_unpack_slots(xlo_ref[...]), _unpack_slots(xhi_ref[...])], axis=1)
  xr = jnp.dot(xj_all, _expand_mat(32, 8),
               preferred_element_type=jnp.float32)
  m_all = _contract_mx(xr * w, 32, 8)
  padded = jnp.concatenate(
      [m_all, jnp.zeros((BE, 8), jnp.float32)], axis=1)
  o_ref[...] = _assemble16(padded)


def _pk_spec():
  return pl.BlockSpec((R256, 128), lambda i: (i, 0))


def _msg_call(body, nxj, kdim, nout):
  return pl.pallas_call(
      body,
      grid=(E // BE,),
      in_specs=[_pk_spec()] * (1 + nxj) + [
          pl.BlockSpec((17, kdim), lambda i: (0, 0))],
      out_specs=[_pk_spec()] * nout if nout > 1 else _pk_spec(),
      out_shape=[jax.ShapeDtypeStruct((E // G, 128), jnp.float32)] * nout
      if nout > 1 else jax.ShapeDtypeStruct((E // G, 128), jnp.float32),
  )


def _combine1_body(p_ref, x_ref, r_ref, b_ref, h_ref, cnt_ref):
  s = p_ref[0, 0] + p_ref[1, 0]
  cnt = p_ref[0, 1, :, 0:1] + p_ref[1, 1, :, 0:1]
  cntc = jnp.maximum(cnt, 1.0)
  root = jnp.dot(x_ref[...], r_ref[...], preferred_element_type=jnp.float32)
  h_ref[...] = jax.nn.relu(s / cntc + root + b_ref[...])
  cnt_ref[...] = cntc


def _combine2_body(p_ref, cnt_ref, h_ref, r_ref, b_ref, olo_ref, ohi_ref):
  s = jnp.concatenate([p_ref[0, 0] + p_ref[1, 0],
                       p_ref[0, 1] + p_ref[1, 1]], axis=1)
  root = jnp.dot(h_ref[...], r_ref[...], preferred_element_type=jnp.float32)
  h2 = jax.nn.relu(s / cnt_ref[...] + root + b_ref[...])
  olo_ref[...] = h2[:, 0:16]
  ohi_ref[...] = h2[:, 16:32]


def _combine3_body(p_ref, cnt_ref, hlo_ref, hhi_ref, r_ref, b_ref,
                   o_ref, ot_ref):
  s = p_ref[0, :, 0:8] + p_ref[1, :, 0:8]
  h2 = jnp.concatenate([hlo_ref[...], hhi_ref[...]], axis=1)
  root = jnp.dot(h2, r_ref[...], preferred_element_type=jnp.float32)
  h3 = jax.nn.relu(s / cnt_ref[...] + root + b_ref[...])
  o_ref[...] = h3
  ot_ref[...] = h3.T


def _cbt_body(hblk_ref, ht_ref, o_ref):
  ha = hblk_ref[...]
  ht = ht_ref[...]
  acc = jnp.abs(ht[0:1, :] - ha[:, 0:1])
  for k in range(1, 8):
    acc = acc + jnp.abs(ht[k:k + 1, :] - ha[:, k:k + 1])
  o_ref[...] = acc


TA = 256


def kernel(x, edge_index, edge_attr, lin1_W, lin1_b, root1, bias1,
           lin2_W, lin2_b, root2, bias2, lin3_W, lin3_b, root3, bias3):
  f32 = jnp.float32
  ei = edge_index.reshape(2, NW, NCH, CH)
  x16 = jnp.broadcast_to(x, (N, 16))
  ap = edge_attr.reshape(E // G, 128)

  gather16 = _make_gather16()
  gather_pair = _make_gather_pair()
  scatter_pair = _make_scatter_pair()
  scatter16 = _make_scatter16()

  waug1 = jnp.concatenate([lin1_W, lin1_b[None, :]], axis=0)
  waug2 = jnp.concatenate([lin2_W, lin2_b[None, :]], axis=0)
  waug3 = jnp.concatenate([lin3_W, lin3_b[None, :]], axis=0)

  xj1 = gather16(x16, ei)
  m1lo, m1hi = _msg_call(_msg1_body, 1, 16, 2)(
      ap, xj1.reshape(E // G, 128), waug1)
  p1 = scatter_pair(m1lo.reshape(E, 16), m1hi.reshape(E, 16), ei)
  h1, cntc = pl.pallas_call(
      _combine1_body,
      in_specs=[pl.BlockSpec((NC, 2, N, 16), lambda: (0, 0, 0, 0)),
                pl.BlockSpec((N, 1), lambda: (0, 0)),
                pl.BlockSpec((1, 16), lambda: (0, 0)),
                pl.BlockSpec((1, 16), lambda: (0, 0))],
      out_specs=[pl.BlockSpec((N, 16), lambda: (0, 0)),
                 pl.BlockSpec((N, 1), lambda: (0, 0))],
      out_shape=[jax.ShapeDtypeStruct((N, 16), f32),
                 jax.ShapeDtypeStruct((N, 1), f32)],
  )(p1, x, root1, bias1.reshape(1, 16))

  xj2 = gather16(h1, ei)
  m2lo, m2hi = _msg_call(_msg2_body, 1, 16 * 32, 2)(
      ap, xj2.reshape(E // G, 128), waug2)
  p2 = scatter_pair(m2lo.reshape(E, 16), m2hi.reshape(E, 16), ei)
  h2lo, h2hi = pl.pallas_call(
      _combine2_body,
      in_specs=[pl.BlockSpec((NC, 2, N, 16), lambda: (0, 0, 0, 0)),
                pl.BlockSpec((N, 1), lambda: (0, 0)),
                pl.BlockSpec((N, 16), lambda: (0, 0)),
                pl.BlockSpec((16, 32), lambda: (0, 0)),
                pl.BlockSpec((1, 32), lambda: (0, 0))],
      out_specs=[pl.BlockSpec((N, 16), lambda: (0, 0)),
                 pl.BlockSpec((N, 16), lambda: (0, 0))],
      out_shape=[jax.ShapeDtypeStruct((N, 16), f32),
                 jax.ShapeDtypeStruct((N, 16), f32)],
  )(p2, cntc, h1, root2, bias2.reshape(1, 32))

  xj3lo, xj3hi = gather_pair(h2lo, h2hi, ei)
  msg3 = _msg_call(_msg3_body, 2, 16 * 16, 1)(
      ap, xj3lo.reshape(E // G, 128), xj3hi.reshape(E // G, 128), waug3)
  p3 = scatter16(msg3.reshape(E, 16), ei)
  h3, h3t = pl.pallas_call(
      _combine3_body,
      in_specs=[pl.BlockSpec((NC, N, 16), lambda: (0, 0, 0)),
                pl.BlockSpec((N, 1), lambda: (0, 0)),
                pl.BlockSpec((N, 16), lambda: (0, 0)),
                pl.BlockSpec((N, 16), lambda: (0, 0)),
                pl.BlockSpec((32, 8), lambda: (0, 0)),
                pl.BlockSpec((1, 8), lambda: (0, 0))],
      out_specs=[pl.BlockSpec((N, 8), lambda: (0, 0)),
                 pl.BlockSpec((8, N), lambda: (0, 0))],
      out_shape=[jax.ShapeDtypeStruct((N, 8), f32),
                 jax.ShapeDtypeStruct((8, N), f32)],
  )(p3, cntc, h2lo, h2hi, root3, bias3.reshape(1, 8))

  cbt = pl.pallas_call(
      _cbt_body,
      grid=(N // TA,),
      in_specs=[pl.BlockSpec((TA, 8), lambda i: (i, 0)),
                pl.BlockSpec((8, N), lambda i: (0, 0))],
      out_specs=pl.BlockSpec((TA, N), lambda i: (i, 0)),
      out_shape=jax.ShapeDtypeStruct((N, N), f32),
  )(h3, h3t)
  return cbt
